# Initial kernel scaffold; baseline (speedup 1.0000x reference)
import jax
import jax.numpy as jnp
from jax.experimental import pallas as pl


def kernel(u1, u2, length_1, length_2, loc_1, loc_2, time_1, time_2, time_gap_1, time_gap_2, loc_table, time_table, W_ih, W_hh, b_ih, b_hh, time_v, time_w, time_b, gat_embedding, fc_W, fc_b):
    raise NotImplementedError("write your pallas kernel here")


# speedup vs baseline: 4.8972x; 4.8972x over previous
import functools

import jax
import jax.numpy as jnp
from jax import lax
from jax.experimental import pallas as pl
from jax.experimental.pallas import tpu as pltpu
from jax.experimental.pallas import tpu_sc as plsc

B = 256
S = 200
ED = 64
H = 128
B2 = 2 * B

NW = 32
ROWS_PER_W = 2 * B * S // NW
CHUNK = 800
U_PER_W = B2 // NW


def _sc_gather(loc_idx, time_idx, u_idx, loc_table, time_table, gat_embedding):
    n = loc_idx.shape[0]
    mesh = plsc.VectorSubcoreMesh(core_axis_name="c", subcore_axis_name="s")

    @functools.partial(
        pl.kernel,
        out_type=[
            jax.ShapeDtypeStruct((n, ED), jnp.float32),
            jax.ShapeDtypeStruct((n, ED), jnp.float32),
            jax.ShapeDtypeStruct((B2, ED), jnp.float32),
        ],
        mesh=mesh,
        scratch_types=[
            pltpu.VMEM((CHUNK,), jnp.int32),
            pltpu.VMEM((CHUNK, ED), jnp.float32),
            pltpu.VMEM((U_PER_W,), jnp.int32),
            pltpu.VMEM((U_PER_W, ED), jnp.float32),
            pltpu.SemaphoreType.DMA,
        ],
        compiler_params=pltpu.CompilerParams(use_tc_tiling_on_sc=False),
    )
    def gather_kernel(loc_idx_h, time_idx_h, u_idx_h, loc_t_h, time_t_h, gat_h,
                      loc_out, time_out, u_out, idx_v, rows_v, uidx_v, urows_v,
                      sem):
        wid = lax.axis_index("s") * 2 + lax.axis_index("c")
        for idx_h, tab_h, out_h in ((loc_idx_h, loc_t_h, loc_out),
                                    (time_idx_h, time_t_h, time_out)):
            for j in range(ROWS_PER_W // CHUNK):
                base = wid * ROWS_PER_W + j * CHUNK
                pltpu.sync_copy(idx_h.at[pl.ds(base, CHUNK)], idx_v)
                pltpu.async_copy(tab_h.at[idx_v], rows_v, sem).wait()
                pltpu.sync_copy(rows_v, out_h.at[pl.ds(base, CHUNK)])
        ubase = wid * U_PER_W
        pltpu.sync_copy(u_idx_h.at[pl.ds(ubase, U_PER_W)], uidx_v)
        pltpu.async_copy(gat_h.at[uidx_v], urows_v, sem).wait()
        pltpu.sync_copy(urows_v, u_out.at[pl.ds(ubase, U_PER_W)])

    return gather_kernel(loc_idx, time_idx, u_idx, loc_table, time_table,
                         gat_embedding)


def _cos_topmax(length_1, length_2, emb1, emb2):
    def body(l1_ref, l2_ref, e1_ref, e2_ref, hang_ref, lie_ref):
        b = pl.program_id(0)
        e1 = e1_ref[0]
        e2 = e2_ref[0]
        r1 = lax.rsqrt(jnp.sum(e1 * e1, axis=1, keepdims=True))
        r2 = lax.rsqrt(jnp.sum(e2 * e2, axis=1, keepdims=True))
        cos = lax.dot_general(e1 * r1, e2 * r2, (((1,), (1,)), ((), ())),
                              preferred_element_type=jnp.float32)
        l1 = l1_ref[b]
        l2 = l2_ref[b]
        row_id = lax.broadcasted_iota(jnp.int32, (S, S), 0)
        col_id = lax.broadcasted_iota(jnp.int32, (S, S), 1)
        neg = jnp.float32(-jnp.inf)
        hang_full = jnp.max(jnp.where(col_id < l2, cos, neg), axis=1,
                            keepdims=True)
        rmask = lax.broadcasted_iota(jnp.int32, (S, 1), 0) < l1
        hang_ref[0] = jnp.where(rmask, hang_full, 0.0)
        lie_full = jnp.max(jnp.where(row_id < l1, cos, neg), axis=0,
                           keepdims=True)
        cmask = lax.broadcasted_iota(jnp.int32, (1, S), 1) < l2
        lie_ref[0] = jnp.where(cmask, lie_full, 0.0)

    return pl.pallas_call(
        body,
        grid=(B,),
        in_specs=[
            pl.BlockSpec(memory_space=pltpu.SMEM),
            pl.BlockSpec(memory_space=pltpu.SMEM),
            pl.BlockSpec((1, S, ED), lambda b: (b, 0, 0)),
            pl.BlockSpec((1, S, ED), lambda b: (b, 0, 0)),
        ],
        out_specs=[
            pl.BlockSpec((1, S, 1), lambda b: (b, 0, 0)),
            pl.BlockSpec((1, 1, S), lambda b: (b, 0, 0)),
        ],
        out_shape=[
            jax.ShapeDtypeStruct((B, S, 1), jnp.float32),
            jax.ShapeDtypeStruct((B, 1, S), jnp.float32),
        ],
    )(length_1, length_2, emb1, emb2)


def _lstm_loss(w_s, b_s, te, tg, lens, W_ihT, W_hhT, b_tot, tv_row):
    def body(w_ref, b0_ref, te_ref, tg_ref, len_ref, wih_ref, whh_ref, bt_ref,
             tv_ref, seq_out, fs_out, h_s, c_s, seq_s, fs_s):
        step = pl.program_id(0)

        @pl.when(step == 0)
        def _():
            h_s[...] = jnp.zeros_like(h_s)
            c_s[...] = jnp.zeros_like(c_s)
            seq_s[...] = jnp.zeros_like(seq_s)
            fs_s[...] = jnp.zeros_like(fs_s)

        x = te_ref[0]
        gates = (lax.dot_general(x, wih_ref[...], (((1,), (0,)), ((), ())),
                                 preferred_element_type=jnp.float32)
                 + lax.dot_general(h_s[...], whh_ref[...],
                                   (((1,), (0,)), ((), ())),
                                   preferred_element_type=jnp.float32)
                 + bt_ref[...])
        gi = jax.nn.sigmoid(gates[:, 0:H])
        gf = jax.nn.sigmoid(gates[:, H:2 * H])
        gg = jnp.tanh(gates[:, 2 * H:3 * H])
        go = jax.nn.sigmoid(gates[:, 3 * H:4 * H])
        c = gf * c_s[...] + gi * gg
        h = go * jnp.tanh(c)
        h_s[...] = h
        c_s[...] = c

        p1 = jnp.sum(h * tv_ref[...], axis=1, keepdims=True)
        w = w_ref[0, 0]
        b0 = b0_ref[0, 0]
        p2 = w * tg_ref[0]
        f1 = p1 + p2 + b0 + (jnp.exp(p1 + b0) - jnp.exp(p1 + p2 + b0)) / w
        lm1 = len_ref[...] - 1
        fs_s[...] = fs_s[...] + jnp.where(step < lm1, f1, 0.0)
        seq_s[...] = jnp.where(lm1 == step, h, seq_s[...])

        @pl.when(step == S - 1)
        def _():
            seq_out[...] = seq_s[...]
            fs_out[...] = fs_s[...]

    return pl.pallas_call(
        body,
        grid=(S,),
        in_specs=[
            pl.BlockSpec(memory_space=pltpu.SMEM),
            pl.BlockSpec(memory_space=pltpu.SMEM),
            pl.BlockSpec((1, B2, ED), lambda s: (s, 0, 0)),
            pl.BlockSpec((1, B2, 1), lambda s: (s, 0, 0)),
            pl.BlockSpec((B2, 1), lambda s: (0, 0)),
            pl.BlockSpec((ED, 4 * H), lambda s: (0, 0)),
            pl.BlockSpec((H, 4 * H), lambda s: (0, 0)),
            pl.BlockSpec((1, 4 * H), lambda s: (0, 0)),
            pl.BlockSpec((1, H), lambda s: (0, 0)),
        ],
        out_specs=[
            pl.BlockSpec((B2, H), lambda s: (0, 0)),
            pl.BlockSpec((B2, 1), lambda s: (0, 0)),
        ],
        out_shape=[
            jax.ShapeDtypeStruct((B2, H), jnp.float32),
            jax.ShapeDtypeStruct((B2, 1), jnp.float32),
        ],
        scratch_shapes=[
            pltpu.VMEM((B2, H), jnp.float32),
            pltpu.VMEM((B2, H), jnp.float32),
            pltpu.VMEM((B2, H), jnp.float32),
            pltpu.VMEM((B2, 1), jnp.float32),
        ],
    )(w_s, b_s, te, tg, lens, W_ihT, W_hhT, b_tot, tv_row)


def _final(hang, lie, seq, u_emb, fsum, lens, w_hang, w_lie, w_time, w_u, fcb):
    def body(hang_ref, lie_ref, seq_ref, u_ref, fs_ref, len_ref, wh_ref,
             wl_ref, wt_ref, wu_ref, fb_ref, out_ref, tl_ref):
        th = jnp.tanh(seq_ref[0:B, :] * seq_ref[B:B2, :])
        tu = jnp.tanh(u_ref[0:B, :] * u_ref[B:B2, :])

        def dn(a, w):
            return lax.dot_general(a, w, (((1,), (0,)), ((), ())),
                                   preferred_element_type=jnp.float32)

        out_ref[...] = (dn(hang_ref[...], wh_ref[...])
                        + dn(lie_ref[...], wl_ref[...])
                        + dn(th, wt_ref[...])
                        + dn(tu, wu_ref[...])
                        + fb_ref[...])
        r = fs_ref[...] / (len_ref[...] - 1).astype(jnp.float32)
        tl_ref[...] = (-jnp.sum(r) / B).reshape(1, 1)

    return pl.pallas_call(
        body,
        out_shape=[
            jax.ShapeDtypeStruct((B, 2), jnp.float32),
            jax.ShapeDtypeStruct((1, 1), jnp.float32),
        ],
    )(hang, lie, seq, u_emb, fsum, lens, w_hang, w_lie, w_time, w_u, fcb)


def kernel(u1, u2, length_1, length_2, loc_1, loc_2, time_1, time_2,
           time_gap_1, time_gap_2, loc_table, time_table, W_ih, W_hh, b_ih,
           b_hh, time_v, time_w, time_b, gat_embedding, fc_W, fc_b):
    loc_idx = jnp.concatenate([loc_1.reshape(-1),
                               loc_2.reshape(-1)]).astype(jnp.int32)
    t_all = jnp.concatenate([time_1, time_2], axis=0)
    time_idx = t_all.T.reshape(-1).astype(jnp.int32)
    u_idx = jnp.concatenate([u1, u2]).astype(jnp.int32)

    emb_loc, emb_time, emb_u = _sc_gather(loc_idx, time_idx, u_idx,
                                          loc_table, time_table,
                                          gat_embedding)

    emb1 = emb_loc[:B * S].reshape(B, S, ED)
    emb2 = emb_loc[B * S:].reshape(B, S, ED)
    l1 = length_1.astype(jnp.int32)
    l2 = length_2.astype(jnp.int32)
    hang3, lie3 = _cos_topmax(l1, l2, emb1, emb2)

    te = emb_time.reshape(S, B2, ED)
    zero_col = jnp.zeros((B, 1), time_gap_1.dtype)
    tg1 = jnp.concatenate([time_gap_1[:, 1:], zero_col], axis=1)
    tg2 = jnp.concatenate([time_gap_2[:, 1:], zero_col], axis=1)
    tg = jnp.concatenate([tg1, tg2], axis=0).T.reshape(S, B2, 1)
    lens = jnp.concatenate([l1, l2]).reshape(B2, 1)
    seq, fsum = _lstm_loss(time_w, time_b, te, tg, lens, W_ih.T, W_hh.T,
                           (b_ih + b_hh).reshape(1, 4 * H),
                           time_v.reshape(1, H))

    outs, tl = _final(hang3.reshape(B, S), lie3.reshape(B, S), seq, emb_u,
                      fsum, lens, fc_W[0:S], fc_W[S:2 * S],
                      fc_W[2 * S:2 * S + H], fc_W[2 * S + H:],
                      fc_b.reshape(1, 2))
    return (outs, tl.reshape(()))


# speedup vs baseline: 4.9421x; 1.0092x over previous
import functools

import jax
import jax.numpy as jnp
from jax import lax
from jax.experimental import pallas as pl
from jax.experimental.pallas import tpu as pltpu
from jax.experimental.pallas import tpu_sc as plsc

B = 256
S = 200
ED = 64
H = 128
B2 = 2 * B

NW = 32
ROWS_PER_W = 2 * B * S // NW
CHUNK = 800
U_PER_W = B2 // NW


def _gather_chunks(idx_h, tab_h, out_h, wid, idx_v, rows_v, gsems, wsems):
    nch = ROWS_PER_W // CHUNK
    gh = [None, None]
    wh = [None, None]
    for j in range(nch + 1):
        if j < nch:
            bb = j % 2
            if wh[bb] is not None:
                wh[bb].wait()
            base = wid * ROWS_PER_W + j * CHUNK
            pltpu.sync_copy(idx_h.at[pl.ds(base, CHUNK)], idx_v[bb])
            gh[bb] = pltpu.async_copy(tab_h.at[idx_v[bb]], rows_v[bb],
                                      gsems[bb])
        if j >= 1:
            pb = (j - 1) % 2
            gh[pb].wait()
            pbase = wid * ROWS_PER_W + (j - 1) * CHUNK
            wh[pb] = pltpu.async_copy(rows_v[pb],
                                      out_h.at[pl.ds(pbase, CHUNK)],
                                      wsems[pb])
    for h in wh:
        if h is not None:
            h.wait()


def _sc_gather_time(time_idx, time_table):
    n = time_idx.shape[0]
    mesh = plsc.VectorSubcoreMesh(core_axis_name="c", subcore_axis_name="s")

    @functools.partial(
        pl.kernel,
        out_type=jax.ShapeDtypeStruct((n, ED), jnp.float32),
        mesh=mesh,
        scratch_types=[
            [pltpu.VMEM((CHUNK,), jnp.int32)] * 2,
            [pltpu.VMEM((CHUNK, ED), jnp.float32)] * 2,
            [pltpu.SemaphoreType.DMA] * 2,
            [pltpu.SemaphoreType.DMA] * 2,
        ],
        compiler_params=pltpu.CompilerParams(use_tc_tiling_on_sc=False),
    )
    def gather_kernel(time_idx_h, time_t_h, time_out, idx_v, rows_v, gsems,
                      wsems):
        wid = lax.axis_index("s") * 2 + lax.axis_index("c")
        _gather_chunks(time_idx_h, time_t_h, time_out, wid, idx_v, rows_v,
                       gsems, wsems)

    return gather_kernel(time_idx, time_table)


def _sc_gather_loc_user(loc_idx, u_idx, loc_table, gat_embedding):
    n = loc_idx.shape[0]
    mesh = plsc.VectorSubcoreMesh(core_axis_name="c", subcore_axis_name="s")

    @functools.partial(
        pl.kernel,
        out_type=[
            jax.ShapeDtypeStruct((n, ED), jnp.float32),
            jax.ShapeDtypeStruct((B2, ED), jnp.float32),
        ],
        mesh=mesh,
        scratch_types=[
            [pltpu.VMEM((CHUNK,), jnp.int32)] * 2,
            [pltpu.VMEM((CHUNK, ED), jnp.float32)] * 2,
            pltpu.VMEM((U_PER_W,), jnp.int32),
            pltpu.VMEM((U_PER_W, ED), jnp.float32),
            [pltpu.SemaphoreType.DMA] * 2,
            [pltpu.SemaphoreType.DMA] * 2,
            pltpu.SemaphoreType.DMA,
        ],
        compiler_params=pltpu.CompilerParams(use_tc_tiling_on_sc=False),
    )
    def gather_kernel(loc_idx_h, u_idx_h, loc_t_h, gat_h, loc_out, u_out,
                      idx_v, rows_v, uidx_v, urows_v, gsems, wsems, usem):
        wid = lax.axis_index("s") * 2 + lax.axis_index("c")
        ubase = wid * U_PER_W
        pltpu.sync_copy(u_idx_h.at[pl.ds(ubase, U_PER_W)], uidx_v)
        uh = pltpu.async_copy(gat_h.at[uidx_v], urows_v, usem)
        _gather_chunks(loc_idx_h, loc_t_h, loc_out, wid, idx_v, rows_v,
                       gsems, wsems)
        uh.wait()
        pltpu.sync_copy(urows_v, u_out.at[pl.ds(ubase, U_PER_W)])

    return gather_kernel(loc_idx, u_idx, loc_table, gat_embedding)


def _cos_topmax(length_1, length_2, emb1, emb2):
    def body(l1_ref, l2_ref, e1_ref, e2_ref, hang_ref, lie_ref):
        b = pl.program_id(0)
        e1 = e1_ref[0]
        e2 = e2_ref[0]
        r1 = lax.rsqrt(jnp.sum(e1 * e1, axis=1, keepdims=True))
        r2 = lax.rsqrt(jnp.sum(e2 * e2, axis=1, keepdims=True))
        cos = lax.dot_general(e1 * r1, e2 * r2, (((1,), (1,)), ((), ())),
                              preferred_element_type=jnp.float32)
        l1 = l1_ref[b]
        l2 = l2_ref[b]
        row_id = lax.broadcasted_iota(jnp.int32, (S, S), 0)
        col_id = lax.broadcasted_iota(jnp.int32, (S, S), 1)
        neg = jnp.float32(-jnp.inf)
        hang_full = jnp.max(jnp.where(col_id < l2, cos, neg), axis=1,
                            keepdims=True)
        rmask = lax.broadcasted_iota(jnp.int32, (S, 1), 0) < l1
        hang_ref[0] = jnp.where(rmask, hang_full, 0.0)
        lie_full = jnp.max(jnp.where(row_id < l1, cos, neg), axis=0,
                           keepdims=True)
        cmask = lax.broadcasted_iota(jnp.int32, (1, S), 1) < l2
        lie_ref[0] = jnp.where(cmask, lie_full, 0.0)

    return pl.pallas_call(
        body,
        grid=(B,),
        in_specs=[
            pl.BlockSpec(memory_space=pltpu.SMEM),
            pl.BlockSpec(memory_space=pltpu.SMEM),
            pl.BlockSpec((1, S, ED), lambda b: (b, 0, 0)),
            pl.BlockSpec((1, S, ED), lambda b: (b, 0, 0)),
        ],
        out_specs=[
            pl.BlockSpec((1, S, 1), lambda b: (b, 0, 0)),
            pl.BlockSpec((1, 1, S), lambda b: (b, 0, 0)),
        ],
        out_shape=[
            jax.ShapeDtypeStruct((B, S, 1), jnp.float32),
            jax.ShapeDtypeStruct((B, 1, S), jnp.float32),
        ],
    )(length_1, length_2, emb1, emb2)


def _lstm_loss(w_s, b_s, te, tg, lens, W_ihT, W_hhT, b_tot, tv_row):
    def body(w_ref, b0_ref, te_ref, tg_ref, len_ref, wih_ref, whh_ref, bt_ref,
             tv_ref, seq_out, fs_out, h_s, c_s, seq_s, fs_s):
        step = pl.program_id(0)

        @pl.when(step == 0)
        def _():
            h_s[...] = jnp.zeros_like(h_s)
            c_s[...] = jnp.zeros_like(c_s)
            seq_s[...] = jnp.zeros_like(seq_s)
            fs_s[...] = jnp.zeros_like(fs_s)

        x = te_ref[0]
        gates = (lax.dot_general(x, wih_ref[...], (((1,), (0,)), ((), ())),
                                 preferred_element_type=jnp.float32)
                 + lax.dot_general(h_s[...], whh_ref[...],
                                   (((1,), (0,)), ((), ())),
                                   preferred_element_type=jnp.float32)
                 + bt_ref[...])
        gi = jax.nn.sigmoid(gates[:, 0:H])
        gf = jax.nn.sigmoid(gates[:, H:2 * H])
        gg = jnp.tanh(gates[:, 2 * H:3 * H])
        go = jax.nn.sigmoid(gates[:, 3 * H:4 * H])
        c = gf * c_s[...] + gi * gg
        h = go * jnp.tanh(c)
        h_s[...] = h
        c_s[...] = c

        p1 = jnp.sum(h * tv_ref[...], axis=1, keepdims=True)
        w = w_ref[0, 0]
        b0 = b0_ref[0, 0]
        p2 = w * tg_ref[0]
        f1 = p1 + p2 + b0 + (jnp.exp(p1 + b0) - jnp.exp(p1 + p2 + b0)) / w
        lm1 = len_ref[...] - 1
        fs_s[...] = fs_s[...] + jnp.where(step < lm1, f1, 0.0)
        seq_s[...] = jnp.where(lm1 == step, h, seq_s[...])

        @pl.when(step == S - 1)
        def _():
            seq_out[...] = seq_s[...]
            fs_out[...] = fs_s[...]

    return pl.pallas_call(
        body,
        grid=(S,),
        in_specs=[
            pl.BlockSpec(memory_space=pltpu.SMEM),
            pl.BlockSpec(memory_space=pltpu.SMEM),
            pl.BlockSpec((1, B2, ED), lambda s: (s, 0, 0)),
            pl.BlockSpec((1, B2, 1), lambda s: (s, 0, 0)),
            pl.BlockSpec((B2, 1), lambda s: (0, 0)),
            pl.BlockSpec((ED, 4 * H), lambda s: (0, 0)),
            pl.BlockSpec((H, 4 * H), lambda s: (0, 0)),
            pl.BlockSpec((1, 4 * H), lambda s: (0, 0)),
            pl.BlockSpec((1, H), lambda s: (0, 0)),
        ],
        out_specs=[
            pl.BlockSpec((B2, H), lambda s: (0, 0)),
            pl.BlockSpec((B2, 1), lambda s: (0, 0)),
        ],
        out_shape=[
            jax.ShapeDtypeStruct((B2, H), jnp.float32),
            jax.ShapeDtypeStruct((B2, 1), jnp.float32),
        ],
        scratch_shapes=[
            pltpu.VMEM((B2, H), jnp.float32),
            pltpu.VMEM((B2, H), jnp.float32),
            pltpu.VMEM((B2, H), jnp.float32),
            pltpu.VMEM((B2, 1), jnp.float32),
        ],
    )(w_s, b_s, te, tg, lens, W_ihT, W_hhT, b_tot, tv_row)


def _final(hang, lie, seq, u_emb, fsum, lens, w_hang, w_lie, w_time, w_u, fcb):
    def body(hang_ref, lie_ref, seq_ref, u_ref, fs_ref, len_ref, wh_ref,
             wl_ref, wt_ref, wu_ref, fb_ref, out_ref, tl_ref):
        th = jnp.tanh(seq_ref[0:B, :] * seq_ref[B:B2, :])
        tu = jnp.tanh(u_ref[0:B, :] * u_ref[B:B2, :])

        def dn(a, w):
            return lax.dot_general(a, w, (((1,), (0,)), ((), ())),
                                   preferred_element_type=jnp.float32)

        out_ref[...] = (dn(hang_ref[...], wh_ref[...])
                        + dn(lie_ref[...], wl_ref[...])
                        + dn(th, wt_ref[...])
                        + dn(tu, wu_ref[...])
                        + fb_ref[...])
        r = fs_ref[...] / (len_ref[...] - 1).astype(jnp.float32)
        tl_ref[...] = (-jnp.sum(r) / B).reshape(1, 1)

    return pl.pallas_call(
        body,
        out_shape=[
            jax.ShapeDtypeStruct((B, 2), jnp.float32),
            jax.ShapeDtypeStruct((1, 1), jnp.float32),
        ],
    )(hang, lie, seq, u_emb, fsum, lens, w_hang, w_lie, w_time, w_u, fcb)


def kernel(u1, u2, length_1, length_2, loc_1, loc_2, time_1, time_2,
           time_gap_1, time_gap_2, loc_table, time_table, W_ih, W_hh, b_ih,
           b_hh, time_v, time_w, time_b, gat_embedding, fc_W, fc_b):
    loc_idx = jnp.concatenate([loc_1.reshape(-1),
                               loc_2.reshape(-1)]).astype(jnp.int32)
    t_all = jnp.concatenate([time_1, time_2], axis=0)
    time_idx = t_all.T.reshape(-1).astype(jnp.int32)
    u_idx = jnp.concatenate([u1, u2]).astype(jnp.int32)

    emb_time = _sc_gather_time(time_idx, time_table)
    emb_loc, emb_u = _sc_gather_loc_user(loc_idx, u_idx, loc_table,
                                         gat_embedding)

    emb1 = emb_loc[:B * S].reshape(B, S, ED)
    emb2 = emb_loc[B * S:].reshape(B, S, ED)
    l1 = length_1.astype(jnp.int32)
    l2 = length_2.astype(jnp.int32)
    hang3, lie3 = _cos_topmax(l1, l2, emb1, emb2)

    te = emb_time.reshape(S, B2, ED)
    zero_col = jnp.zeros((B, 1), time_gap_1.dtype)
    tg1 = jnp.concatenate([time_gap_1[:, 1:], zero_col], axis=1)
    tg2 = jnp.concatenate([time_gap_2[:, 1:], zero_col], axis=1)
    tg = jnp.concatenate([tg1, tg2], axis=0).T.reshape(S, B2, 1)
    lens = jnp.concatenate([l1, l2]).reshape(B2, 1)
    seq, fsum = _lstm_loss(time_w, time_b, te, tg, lens, W_ih.T, W_hh.T,
                           (b_ih + b_hh).reshape(1, 4 * H),
                           time_v.reshape(1, H))

    outs, tl = _final(hang3.reshape(B, S), lie3.reshape(B, S), seq, emb_u,
                      fsum, lens, fc_W[0:S], fc_W[S:2 * S],
                      fc_W[2 * S:2 * S + H], fc_W[2 * S + H:],
                      fc_b.reshape(1, 2))
    return (outs, tl.reshape(()))


# speedup vs baseline: 6.6685x; 1.3493x over previous
import functools

import jax
import jax.numpy as jnp
from jax import lax
from jax.experimental import pallas as pl
from jax.experimental.pallas import tpu as pltpu
from jax.experimental.pallas import tpu_sc as plsc

B = 256
S = 200
ED = 64
H = 128
B2 = 2 * B

NW = 32
ROWS_PER_W = 2 * B * S // NW
CHUNK = 800
U_PER_W = B2 // NW


def _sc_gather(loc_idx, time_idx, u_idx, loc_table, time_table, gat_embedding):
    n = loc_idx.shape[0]
    mesh = plsc.VectorSubcoreMesh(core_axis_name="c", subcore_axis_name="s")

    @functools.partial(
        pl.kernel,
        out_type=[
            jax.ShapeDtypeStruct((n, ED), jnp.float32),
            jax.ShapeDtypeStruct((n, ED), jnp.float32),
            jax.ShapeDtypeStruct((B2, ED), jnp.float32),
        ],
        mesh=mesh,
        scratch_types=[
            [pltpu.VMEM((CHUNK,), jnp.int32)] * 2,
            [pltpu.VMEM((CHUNK, ED), jnp.float32)] * 2,
            pltpu.VMEM((U_PER_W,), jnp.int32),
            pltpu.VMEM((U_PER_W, ED), jnp.float32),
            [pltpu.SemaphoreType.DMA] * 2,
            [pltpu.SemaphoreType.DMA] * 2,
            pltpu.SemaphoreType.DMA,
        ],
        compiler_params=pltpu.CompilerParams(use_tc_tiling_on_sc=False),
    )
    def gather_kernel(loc_idx_h, time_idx_h, u_idx_h, loc_t_h, time_t_h,
                      gat_h, loc_out, time_out, u_out, idx_v, rows_v, uidx_v,
                      urows_v, gsems, wsems, usem):
        wid = lax.axis_index("s") * 2 + lax.axis_index("c")
        ubase = wid * U_PER_W
        pltpu.sync_copy(u_idx_h.at[pl.ds(ubase, U_PER_W)], uidx_v)
        uh = pltpu.async_copy(gat_h.at[uidx_v], urows_v, usem)

        nch = ROWS_PER_W // CHUNK
        work = ([(loc_idx_h, loc_t_h, loc_out, j) for j in range(nch)]
                + [(time_idx_h, time_t_h, time_out, j) for j in range(nch)])
        gh = [None, None]
        wh = [None, None]
        pend = [None, None]
        for j in range(len(work) + 1):
            if j < len(work):
                bb = j % 2
                if wh[bb] is not None:
                    wh[bb].wait()
                idx_h, tab_h, out_h, cj = work[j]
                base = wid * ROWS_PER_W + cj * CHUNK
                pltpu.sync_copy(idx_h.at[pl.ds(base, CHUNK)], idx_v[bb])
                gh[bb] = pltpu.async_copy(tab_h.at[idx_v[bb]], rows_v[bb],
                                          gsems[bb])
                pend[bb] = (out_h, base)
            if j >= 1:
                pb = (j - 1) % 2
                gh[pb].wait()
                out_h, base = pend[pb]
                wh[pb] = pltpu.async_copy(rows_v[pb],
                                          out_h.at[pl.ds(base, CHUNK)],
                                          wsems[pb])
        for h in wh:
            if h is not None:
                h.wait()
        uh.wait()
        pltpu.sync_copy(urows_v, u_out.at[pl.ds(ubase, U_PER_W)])

    return gather_kernel(loc_idx, time_idx, u_idx, loc_table, time_table,
                         gat_embedding)


COS_BATCH = 8


def _cos_topmax(length_1, length_2, emb_loc):
    def body(l1_ref, l2_ref, e1_ref, e2_ref, hang_ref, lie_ref):
        b0 = pl.program_id(0) * COS_BATCH
        neg = jnp.float32(-jnp.inf)
        hang_cols = []
        lie_rows = []
        for i in range(COS_BATCH):
            e1 = e1_ref[pl.ds(i * S, S), :]
            e2 = e2_ref[pl.ds(i * S, S), :]
            r1 = lax.rsqrt(jnp.sum(e1 * e1, axis=1, keepdims=True))
            r2 = lax.rsqrt(jnp.sum(e2 * e2, axis=1, keepdims=True))
            cos = lax.dot_general(e1 * r1, e2 * r2, (((1,), (1,)), ((), ())),
                                  preferred_element_type=jnp.float32)
            l1 = l1_ref[b0 + i]
            l2 = l2_ref[b0 + i]
            row_id = lax.broadcasted_iota(jnp.int32, (S, S), 0)
            col_id = lax.broadcasted_iota(jnp.int32, (S, S), 1)
            hang_full = jnp.max(jnp.where(col_id < l2, cos, neg), axis=1,
                                keepdims=True)
            rmask = lax.broadcasted_iota(jnp.int32, (S, 1), 0) < l1
            hang_cols.append(jnp.where(rmask, hang_full, 0.0))
            lie_full = jnp.max(jnp.where(row_id < l1, cos, neg), axis=0,
                               keepdims=True)
            cmask = lax.broadcasted_iota(jnp.int32, (1, S), 1) < l2
            lie_rows.append(jnp.where(cmask, lie_full, 0.0))
        hang_ref[...] = jnp.concatenate(hang_cols, axis=1).T
        lie_ref[...] = jnp.concatenate(lie_rows, axis=0)

    return pl.pallas_call(
        body,
        grid=(B // COS_BATCH,),
        in_specs=[
            pl.BlockSpec(memory_space=pltpu.SMEM),
            pl.BlockSpec(memory_space=pltpu.SMEM),
            pl.BlockSpec((COS_BATCH * S, ED), lambda b: (b, 0)),
            pl.BlockSpec((COS_BATCH * S, ED), lambda b: (b + B // COS_BATCH, 0)),
        ],
        out_specs=[
            pl.BlockSpec((COS_BATCH, S), lambda b: (b, 0)),
            pl.BlockSpec((COS_BATCH, S), lambda b: (b, 0)),
        ],
        out_shape=[
            jax.ShapeDtypeStruct((B, S), jnp.float32),
            jax.ShapeDtypeStruct((B, S), jnp.float32),
        ],
    )(length_1, length_2, emb_loc, emb_loc)


def _lstm_loss(w_s, b_s, te, tg, lens, W_ihT, W_hhT, b_tot, tv_row):
    def body(w_ref, b0_ref, te_ref, tg_ref, len_ref, wih_ref, whh_ref, bt_ref,
             tv_ref, seq_out, fs_out, h_s, c_s, seq_s, fs_s):
        step = pl.program_id(0)

        @pl.when(step == 0)
        def _():
            h_s[...] = jnp.zeros_like(h_s)
            c_s[...] = jnp.zeros_like(c_s)
            seq_s[...] = jnp.zeros_like(seq_s)
            fs_s[...] = jnp.zeros_like(fs_s)

        x = te_ref[...]
        gates = (lax.dot_general(x, wih_ref[...], (((1,), (0,)), ((), ())),
                                 preferred_element_type=jnp.float32)
                 + lax.dot_general(h_s[...], whh_ref[...],
                                   (((1,), (0,)), ((), ())),
                                   preferred_element_type=jnp.float32)
                 + bt_ref[...])
        gi = jax.nn.sigmoid(gates[:, 0:H])
        gf = jax.nn.sigmoid(gates[:, H:2 * H])
        gg = jnp.tanh(gates[:, 2 * H:3 * H])
        go = jax.nn.sigmoid(gates[:, 3 * H:4 * H])
        c = gf * c_s[...] + gi * gg
        h = go * jnp.tanh(c)
        h_s[...] = h
        c_s[...] = c

        p1 = jnp.sum(h * tv_ref[...], axis=1, keepdims=True)
        w = w_ref[0, 0]
        b0 = b0_ref[0, 0]
        p2 = w * tg_ref[0]
        f1 = p1 + p2 + b0 + (jnp.exp(p1 + b0) - jnp.exp(p1 + p2 + b0)) / w
        lm1 = len_ref[...] - 1
        fs_s[...] = fs_s[...] + jnp.where(step < lm1, f1, 0.0)
        seq_s[...] = jnp.where(lm1 == step, h, seq_s[...])

        @pl.when(step == S - 1)
        def _():
            seq_out[...] = seq_s[...]
            fs_out[...] = fs_s[...]

    return pl.pallas_call(
        body,
        grid=(S,),
        in_specs=[
            pl.BlockSpec(memory_space=pltpu.SMEM),
            pl.BlockSpec(memory_space=pltpu.SMEM),
            pl.BlockSpec((B2, ED), lambda s: (s, 0)),
            pl.BlockSpec((1, B2, 1), lambda s: (s, 0, 0)),
            pl.BlockSpec((B2, 1), lambda s: (0, 0)),
            pl.BlockSpec((ED, 4 * H), lambda s: (0, 0)),
            pl.BlockSpec((H, 4 * H), lambda s: (0, 0)),
            pl.BlockSpec((1, 4 * H), lambda s: (0, 0)),
            pl.BlockSpec((1, H), lambda s: (0, 0)),
        ],
        out_specs=[
            pl.BlockSpec((B2, H), lambda s: (0, 0)),
            pl.BlockSpec((B2, 1), lambda s: (0, 0)),
        ],
        out_shape=[
            jax.ShapeDtypeStruct((B2, H), jnp.float32),
            jax.ShapeDtypeStruct((B2, 1), jnp.float32),
        ],
        scratch_shapes=[
            pltpu.VMEM((B2, H), jnp.float32),
            pltpu.VMEM((B2, H), jnp.float32),
            pltpu.VMEM((B2, H), jnp.float32),
            pltpu.VMEM((B2, 1), jnp.float32),
        ],
    )(w_s, b_s, te, tg, lens, W_ihT, W_hhT, b_tot, tv_row)


def _final(hang, lie, seq, u_emb, fsum, lens, w_hang, w_lie, w_time, w_u, fcb):
    def body(hang_ref, lie_ref, seq_ref, u_ref, fs_ref, len_ref, wh_ref,
             wl_ref, wt_ref, wu_ref, fb_ref, out_ref, tl_ref):
        th = jnp.tanh(seq_ref[0:B, :] * seq_ref[B:B2, :])
        tu = jnp.tanh(u_ref[0:B, :] * u_ref[B:B2, :])

        def dn(a, w):
            return lax.dot_general(a, w, (((1,), (0,)), ((), ())),
                                   preferred_element_type=jnp.float32)

        out_ref[...] = (dn(hang_ref[...], wh_ref[...])
                        + dn(lie_ref[...], wl_ref[...])
                        + dn(th, wt_ref[...])
                        + dn(tu, wu_ref[...])
                        + fb_ref[...])
        r = fs_ref[...] / (len_ref[...] - 1).astype(jnp.float32)
        tl_ref[...] = (-jnp.sum(r) / B).reshape(1, 1)

    return pl.pallas_call(
        body,
        out_shape=[
            jax.ShapeDtypeStruct((B, 2), jnp.float32),
            jax.ShapeDtypeStruct((1, 1), jnp.float32),
        ],
    )(hang, lie, seq, u_emb, fsum, lens, w_hang, w_lie, w_time, w_u, fcb)


def kernel(u1, u2, length_1, length_2, loc_1, loc_2, time_1, time_2,
           time_gap_1, time_gap_2, loc_table, time_table, W_ih, W_hh, b_ih,
           b_hh, time_v, time_w, time_b, gat_embedding, fc_W, fc_b):
    loc_idx = jnp.concatenate([loc_1.reshape(-1),
                               loc_2.reshape(-1)]).astype(jnp.int32)
    t_all = jnp.concatenate([time_1, time_2], axis=0)
    time_idx = t_all.T.reshape(-1).astype(jnp.int32)
    u_idx = jnp.concatenate([u1, u2]).astype(jnp.int32)

    emb_loc, emb_time, emb_u = _sc_gather(loc_idx, time_idx, u_idx,
                                          loc_table, time_table,
                                          gat_embedding)

    l1 = length_1.astype(jnp.int32)
    l2 = length_2.astype(jnp.int32)
    hang, lie = _cos_topmax(l1, l2, emb_loc)

    zero_col = jnp.zeros((B, 1), time_gap_1.dtype)
    tg1 = jnp.concatenate([time_gap_1[:, 1:], zero_col], axis=1)
    tg2 = jnp.concatenate([time_gap_2[:, 1:], zero_col], axis=1)
    tg = jnp.concatenate([tg1, tg2], axis=0).T.reshape(S, B2, 1)
    lens = jnp.concatenate([l1, l2]).reshape(B2, 1)
    seq, fsum = _lstm_loss(time_w, time_b, emb_time, tg, lens, W_ih.T, W_hh.T,
                           (b_ih + b_hh).reshape(1, 4 * H),
                           time_v.reshape(1, H))

    outs, tl = _final(hang, lie, seq, emb_u,
                      fsum, lens, fc_W[0:S], fc_W[S:2 * S],
                      fc_W[2 * S:2 * S + H], fc_W[2 * S + H:],
                      fc_b.reshape(1, 2))
    return (outs, tl.reshape(()))


# speedup vs baseline: 6.9863x; 1.0476x over previous
import functools

import jax
import jax.numpy as jnp
from jax import lax
from jax.experimental import pallas as pl
from jax.experimental.pallas import tpu as pltpu
from jax.experimental.pallas import tpu_sc as plsc

B = 256
S = 200
ED = 64
H = 128
B2 = 2 * B

NW = 32
ROWS_PER_W = 2 * B * S // NW
CHUNK = 800
U_PER_W = B2 // NW

_SC_PARAMS = pltpu.CompilerParams(use_tc_tiling_on_sc=False)
_MESH = dict(core_axis_name="c", subcore_axis_name="s")


def _gather_chunks(idx_h, tab_h, out_h, wid, idx_v, rows_v, gsems, wsems):
    nch = ROWS_PER_W // CHUNK
    gh = [None, None]
    wh = [None, None]
    for j in range(nch + 1):
        if j < nch:
            bb = j % 2
            if wh[bb] is not None:
                wh[bb].wait()
            base = wid * ROWS_PER_W + j * CHUNK
            pltpu.sync_copy(idx_h.at[pl.ds(base, CHUNK)], idx_v[bb])
            gh[bb] = pltpu.async_copy(tab_h.at[idx_v[bb]], rows_v[bb],
                                      gsems[bb])
        if j >= 1:
            pb = (j - 1) % 2
            gh[pb].wait()
            pbase = wid * ROWS_PER_W + (j - 1) * CHUNK
            wh[pb] = pltpu.async_copy(rows_v[pb],
                                      out_h.at[pl.ds(pbase, CHUNK)],
                                      wsems[pb])
    for h in wh:
        if h is not None:
            h.wait()


def _sc_gather_time_user(time_idx, u_idx, time_table, gat_embedding):
    n = time_idx.shape[0]

    @functools.partial(
        pl.kernel,
        out_type=[
            jax.ShapeDtypeStruct((n, ED), jnp.float32),
            jax.ShapeDtypeStruct((B2, ED), jnp.float32),
        ],
        mesh=plsc.VectorSubcoreMesh(**_MESH),
        scratch_types=[
            [pltpu.VMEM((CHUNK,), jnp.int32)] * 2,
            [pltpu.VMEM((CHUNK, ED), jnp.float32)] * 2,
            pltpu.VMEM((U_PER_W,), jnp.int32),
            pltpu.VMEM((U_PER_W, ED), jnp.float32),
            [pltpu.SemaphoreType.DMA] * 2,
            [pltpu.SemaphoreType.DMA] * 2,
            pltpu.SemaphoreType.DMA,
        ],
        compiler_params=_SC_PARAMS,
    )
    def gather_kernel(time_idx_h, u_idx_h, time_t_h, gat_h, time_out, u_out,
                      idx_v, rows_v, uidx_v, urows_v, gsems, wsems, usem):
        wid = lax.axis_index("s") * 2 + lax.axis_index("c")
        ubase = wid * U_PER_W
        pltpu.sync_copy(u_idx_h.at[pl.ds(ubase, U_PER_W)], uidx_v)
        uh = pltpu.async_copy(gat_h.at[uidx_v], urows_v, usem)
        _gather_chunks(time_idx_h, time_t_h, time_out, wid, idx_v, rows_v,
                       gsems, wsems)
        uh.wait()
        pltpu.sync_copy(urows_v, u_out.at[pl.ds(ubase, U_PER_W)])

    return gather_kernel(time_idx, u_idx, time_table, gat_embedding)


def _sc_gather_loc(loc_idx, loc_table):
    n = loc_idx.shape[0]

    @functools.partial(
        pl.kernel,
        out_type=jax.ShapeDtypeStruct((n, ED), jnp.float32),
        mesh=plsc.VectorSubcoreMesh(**_MESH),
        scratch_types=[
            [pltpu.VMEM((CHUNK,), jnp.int32)] * 2,
            [pltpu.VMEM((CHUNK, ED), jnp.float32)] * 2,
            [pltpu.SemaphoreType.DMA] * 2,
            [pltpu.SemaphoreType.DMA] * 2,
        ],
        compiler_params=_SC_PARAMS,
    )
    def gather_kernel(loc_idx_h, loc_t_h, loc_out, idx_v, rows_v, gsems,
                      wsems):
        wid = lax.axis_index("s") * 2 + lax.axis_index("c")
        _gather_chunks(loc_idx_h, loc_t_h, loc_out, wid, idx_v, rows_v,
                       gsems, wsems)

    return gather_kernel(loc_idx, loc_table)


COS_BATCH = 8


def _cos_topmax(length_1, length_2, emb_loc):
    def body(l1_ref, l2_ref, e1_ref, e2_ref, hang_ref, lie_ref):
        b0 = pl.program_id(0) * COS_BATCH
        neg = jnp.float32(-jnp.inf)
        hang_cols = []
        lie_rows = []
        for i in range(COS_BATCH):
            e1 = e1_ref[pl.ds(i * S, S), :]
            e2 = e2_ref[pl.ds(i * S, S), :]
            r1 = lax.rsqrt(jnp.sum(e1 * e1, axis=1, keepdims=True))
            r2 = lax.rsqrt(jnp.sum(e2 * e2, axis=1, keepdims=True))
            cos = lax.dot_general(e1 * r1, e2 * r2, (((1,), (1,)), ((), ())),
                                  preferred_element_type=jnp.float32)
            l1 = l1_ref[b0 + i]
            l2 = l2_ref[b0 + i]
            row_id = lax.broadcasted_iota(jnp.int32, (S, S), 0)
            col_id = lax.broadcasted_iota(jnp.int32, (S, S), 1)
            hang_full = jnp.max(jnp.where(col_id < l2, cos, neg), axis=1,
                                keepdims=True)
            rmask = lax.broadcasted_iota(jnp.int32, (S, 1), 0) < l1
            hang_cols.append(jnp.where(rmask, hang_full, 0.0))
            lie_full = jnp.max(jnp.where(row_id < l1, cos, neg), axis=0,
                               keepdims=True)
            cmask = lax.broadcasted_iota(jnp.int32, (1, S), 1) < l2
            lie_rows.append(jnp.where(cmask, lie_full, 0.0))
        hang_ref[...] = jnp.concatenate(hang_cols, axis=1).T
        lie_ref[...] = jnp.concatenate(lie_rows, axis=0)

    return pl.pallas_call(
        body,
        grid=(B // COS_BATCH,),
        in_specs=[
            pl.BlockSpec(memory_space=pltpu.SMEM),
            pl.BlockSpec(memory_space=pltpu.SMEM),
            pl.BlockSpec((COS_BATCH * S, ED), lambda b: (b, 0)),
            pl.BlockSpec((COS_BATCH * S, ED),
                         lambda b: (b + B // COS_BATCH, 0)),
        ],
        out_specs=[
            pl.BlockSpec((COS_BATCH, S), lambda b: (b, 0)),
            pl.BlockSpec((COS_BATCH, S), lambda b: (b, 0)),
        ],
        out_shape=[
            jax.ShapeDtypeStruct((B, S), jnp.float32),
            jax.ShapeDtypeStruct((B, S), jnp.float32),
        ],
    )(length_1, length_2, emb_loc, emb_loc)


def _lstm_loss(w_s, b_s, te, tg, lens, W_ihT, W_hhT, b_tot, tv_row):
    def body(w_ref, b0_ref, te_ref, tg_ref, len_ref, wih_ref, whh_ref, bt_ref,
             tv_ref, seq_out, fs_out, h_s, c_s, seq_s, fs_s):
        step = pl.program_id(0)

        @pl.when(step == 0)
        def _():
            h_s[...] = jnp.zeros_like(h_s)
            c_s[...] = jnp.zeros_like(c_s)
            seq_s[...] = jnp.zeros_like(seq_s)
            fs_s[...] = jnp.zeros_like(fs_s)

        x = te_ref[...].astype(jnp.bfloat16)
        gates = (lax.dot_general(x, wih_ref[...], (((1,), (0,)), ((), ())),
                                 preferred_element_type=jnp.float32)
                 + lax.dot_general(h_s[...].astype(jnp.bfloat16),
                                   whh_ref[...], (((1,), (0,)), ((), ())),
                                   preferred_element_type=jnp.float32)
                 + bt_ref[...])
        gi = jax.nn.sigmoid(gates[:, 0:H])
        gf = jax.nn.sigmoid(gates[:, H:2 * H])
        gg = jnp.tanh(gates[:, 2 * H:3 * H])
        go = jax.nn.sigmoid(gates[:, 3 * H:4 * H])
        c = gf * c_s[...] + gi * gg
        h = go * jnp.tanh(c)
        h_s[...] = h
        c_s[...] = c

        p1 = jnp.sum(h * tv_ref[...], axis=1, keepdims=True)
        w = w_ref[0, 0]
        b0 = b0_ref[0, 0]
        p2 = w * tg_ref[0]
        f1 = p1 + p2 + b0 + (jnp.exp(p1 + b0) - jnp.exp(p1 + p2 + b0)) / w
        lm1 = len_ref[...] - 1
        fs_s[...] = fs_s[...] + jnp.where(step < lm1, f1, 0.0)
        seq_s[...] = jnp.where(lm1 == step, h, seq_s[...])

        @pl.when(step == S - 1)
        def _():
            seq_out[...] = seq_s[...]
            fs_out[...] = fs_s[...]

    return pl.pallas_call(
        body,
        grid=(S,),
        in_specs=[
            pl.BlockSpec(memory_space=pltpu.SMEM),
            pl.BlockSpec(memory_space=pltpu.SMEM),
            pl.BlockSpec((B2, ED), lambda s: (s, 0)),
            pl.BlockSpec((1, B2, 1), lambda s: (s, 0, 0)),
            pl.BlockSpec((B2, 1), lambda s: (0, 0)),
            pl.BlockSpec((ED, 4 * H), lambda s: (0, 0)),
            pl.BlockSpec((H, 4 * H), lambda s: (0, 0)),
            pl.BlockSpec((1, 4 * H), lambda s: (0, 0)),
            pl.BlockSpec((1, H), lambda s: (0, 0)),
        ],
        out_specs=[
            pl.BlockSpec((B2, H), lambda s: (0, 0)),
            pl.BlockSpec((B2, 1), lambda s: (0, 0)),
        ],
        out_shape=[
            jax.ShapeDtypeStruct((B2, H), jnp.float32),
            jax.ShapeDtypeStruct((B2, 1), jnp.float32),
        ],
        scratch_shapes=[
            pltpu.VMEM((B2, H), jnp.float32),
            pltpu.VMEM((B2, H), jnp.float32),
            pltpu.VMEM((B2, H), jnp.float32),
            pltpu.VMEM((B2, 1), jnp.float32),
        ],
    )(w_s, b_s, te, tg, lens, W_ihT, W_hhT, b_tot, tv_row)


def _final(hang, lie, seq, u_emb, fsum, lens, w_hang, w_lie, w_time, w_u,
           fcb):
    def body(hang_ref, lie_ref, seq_ref, u_ref, fs_ref, len_ref, wh_ref,
             wl_ref, wt_ref, wu_ref, fb_ref, out_ref, tl_ref):
        th = jnp.tanh(seq_ref[0:B, :] * seq_ref[B:B2, :])
        tu = jnp.tanh(u_ref[0:B, :] * u_ref[B:B2, :])

        def dn(a, w):
            return lax.dot_general(a, w, (((1,), (0,)), ((), ())),
                                   preferred_element_type=jnp.float32)

        out_ref[...] = (dn(hang_ref[...], wh_ref[...])
                        + dn(lie_ref[...], wl_ref[...])
                        + dn(th, wt_ref[...])
                        + dn(tu, wu_ref[...])
                        + fb_ref[...])
        r = fs_ref[...] / (len_ref[...] - 1).astype(jnp.float32)
        tl_ref[...] = (-jnp.sum(r) / B).reshape(1, 1)

    return pl.pallas_call(
        body,
        out_shape=[
            jax.ShapeDtypeStruct((B, 2), jnp.float32),
            jax.ShapeDtypeStruct((1, 1), jnp.float32),
        ],
    )(hang, lie, seq, u_emb, fsum, lens, w_hang, w_lie, w_time, w_u, fcb)


def kernel(u1, u2, length_1, length_2, loc_1, loc_2, time_1, time_2,
           time_gap_1, time_gap_2, loc_table, time_table, W_ih, W_hh, b_ih,
           b_hh, time_v, time_w, time_b, gat_embedding, fc_W, fc_b):
    t_all = jnp.concatenate([time_1, time_2], axis=0)
    time_idx = t_all.T.reshape(-1).astype(jnp.int32)
    u_idx = jnp.concatenate([u1, u2]).astype(jnp.int32)
    emb_time, emb_u = _sc_gather_time_user(time_idx, u_idx, time_table,
                                           gat_embedding)

    token = (emb_time[0, 0] * 0.0).astype(jnp.int32)
    loc_idx = (jnp.concatenate([loc_1.reshape(-1), loc_2.reshape(-1)])
               .astype(jnp.int32) + token)
    emb_loc = _sc_gather_loc(loc_idx, loc_table)

    l1 = length_1.astype(jnp.int32)
    l2 = length_2.astype(jnp.int32)
    hang, lie = _cos_topmax(l1, l2, emb_loc)

    zero_col = jnp.zeros((B, 1), time_gap_1.dtype)
    tg1 = jnp.concatenate([time_gap_1[:, 1:], zero_col], axis=1)
    tg2 = jnp.concatenate([time_gap_2[:, 1:], zero_col], axis=1)
    tg = jnp.concatenate([tg1, tg2], axis=0).T.reshape(S, B2, 1)
    lens = jnp.concatenate([l1, l2]).reshape(B2, 1)
    seq, fsum = _lstm_loss(time_w, time_b, emb_time, tg, lens,
                           W_ih.T.astype(jnp.bfloat16),
                           W_hh.T.astype(jnp.bfloat16),
                           (b_ih + b_hh).reshape(1, 4 * H),
                           time_v.reshape(1, H))

    outs, tl = _final(hang, lie, seq, emb_u, fsum, lens, fc_W[0:S],
                      fc_W[S:2 * S], fc_W[2 * S:2 * S + H],
                      fc_W[2 * S + H:], fc_b.reshape(1, 2))
    return (outs, tl.reshape(()))


# speedup vs baseline: 8.0682x; 1.1549x over previous
import functools

import jax
import jax.numpy as jnp
from jax import lax
from jax.experimental import pallas as pl
from jax.experimental.pallas import tpu as pltpu
from jax.experimental.pallas import tpu_sc as plsc

B = 256
S = 200
ED = 64
H = 128
B2 = 2 * B
TIMEN = 168

NW = 32
HALF_PER_W = B * S // NW
CHUNK = 800
U_PER_W = B2 // NW


def _sc_gather_loc_user(loc_idx, u_idx, loc_table, gat_embedding):
    @functools.partial(
        pl.kernel,
        out_type=[
            jax.ShapeDtypeStruct((B * S, ED), jnp.float32),
            jax.ShapeDtypeStruct((B * S, ED), jnp.float32),
            jax.ShapeDtypeStruct((B2, ED), jnp.float32),
        ],
        mesh=plsc.VectorSubcoreMesh(core_axis_name="c", subcore_axis_name="s"),
        scratch_types=[
            [pltpu.VMEM((CHUNK,), jnp.int32)] * 2,
            [pltpu.VMEM((CHUNK, ED), jnp.float32)] * 2,
            pltpu.VMEM((U_PER_W,), jnp.int32),
            pltpu.VMEM((U_PER_W, ED), jnp.float32),
            [pltpu.SemaphoreType.DMA] * 2,
            [pltpu.SemaphoreType.DMA] * 2,
            pltpu.SemaphoreType.DMA,
        ],
        compiler_params=pltpu.CompilerParams(use_tc_tiling_on_sc=False),
    )
    def gather_kernel(loc_idx_h, u_idx_h, loc_t_h, gat_h, e1_out, e2_out,
                      u_out, idx_v, rows_v, uidx_v, urows_v, gsems, wsems,
                      usem):
        wid = lax.axis_index("s") * 2 + lax.axis_index("c")
        ubase = wid * U_PER_W
        pltpu.sync_copy(u_idx_h.at[pl.ds(ubase, U_PER_W)], uidx_v)
        uh = pltpu.async_copy(gat_h.at[uidx_v], urows_v, usem)

        nch = HALF_PER_W // CHUNK
        work = ([(0, e1_out, j) for j in range(nch)]
                + [(B * S, e2_out, j) for j in range(nch)])
        gh = [None, None]
        wh = [None, None]
        pend = [None, None]
        for j in range(len(work) + 1):
            if j < len(work):
                bb = j % 2
                if wh[bb] is not None:
                    wh[bb].wait()
                off, out_h, cj = work[j]
                base = wid * HALF_PER_W + cj * CHUNK
                pltpu.sync_copy(loc_idx_h.at[pl.ds(off + base, CHUNK)],
                                idx_v[bb])
                gh[bb] = pltpu.async_copy(loc_t_h.at[idx_v[bb]], rows_v[bb],
                                          gsems[bb])
                pend[bb] = (out_h, base)
            if j >= 1:
                pb = (j - 1) % 2
                gh[pb].wait()
                out_h, base = pend[pb]
                wh[pb] = pltpu.async_copy(rows_v[pb],
                                          out_h.at[pl.ds(base, CHUNK)],
                                          wsems[pb])
        for h in wh:
            if h is not None:
                h.wait()
        uh.wait()
        pltpu.sync_copy(urows_v, u_out.at[pl.ds(ubase, U_PER_W)])

    return gather_kernel(loc_idx, u_idx, loc_table, gat_embedding)


COS_BATCH = 8


def _cos_topmax(length_1, length_2, emb1, emb2):
    def body(l1_ref, l2_ref, e1_ref, e2_ref, hang_ref, lie_ref):
        b0 = pl.program_id(0) * COS_BATCH
        neg = jnp.float32(-jnp.inf)
        hang_cols = []
        lie_rows = []
        for i in range(COS_BATCH):
            e1 = e1_ref[pl.ds(i * S, S), :]
            e2 = e2_ref[pl.ds(i * S, S), :]
            r1 = lax.rsqrt(jnp.sum(e1 * e1, axis=1, keepdims=True))
            r2 = lax.rsqrt(jnp.sum(e2 * e2, axis=1, keepdims=True))
            cos = lax.dot_general(e1 * r1, e2 * r2, (((1,), (1,)), ((), ())),
                                  preferred_element_type=jnp.float32)
            l1 = l1_ref[b0 + i]
            l2 = l2_ref[b0 + i]
            row_id = lax.broadcasted_iota(jnp.int32, (S, S), 0)
            col_id = lax.broadcasted_iota(jnp.int32, (S, S), 1)
            hang_full = jnp.max(jnp.where(col_id < l2, cos, neg), axis=1,
                                keepdims=True)
            rmask = lax.broadcasted_iota(jnp.int32, (S, 1), 0) < l1
            hang_cols.append(jnp.where(rmask, hang_full, 0.0))
            lie_full = jnp.max(jnp.where(row_id < l1, cos, neg), axis=0,
                               keepdims=True)
            cmask = lax.broadcasted_iota(jnp.int32, (1, S), 1) < l2
            lie_rows.append(jnp.where(cmask, lie_full, 0.0))
        hang_ref[...] = jnp.concatenate(hang_cols, axis=1).T
        lie_ref[...] = jnp.concatenate(lie_rows, axis=0)

    return pl.pallas_call(
        body,
        grid=(B // COS_BATCH,),
        in_specs=[
            pl.BlockSpec(memory_space=pltpu.SMEM),
            pl.BlockSpec(memory_space=pltpu.SMEM),
            pl.BlockSpec((COS_BATCH * S, ED), lambda b: (b, 0)),
            pl.BlockSpec((COS_BATCH * S, ED), lambda b: (b, 0)),
        ],
        out_specs=[
            pl.BlockSpec((COS_BATCH, S), lambda b: (b, 0)),
            pl.BlockSpec((COS_BATCH, S), lambda b: (b, 0)),
        ],
        out_shape=[
            jax.ShapeDtypeStruct((B, S), jnp.float32),
            jax.ShapeDtypeStruct((B, S), jnp.float32),
        ],
    )(length_1, length_2, emb1, emb2)


def _lstm_loss(w_s, b_s, times, tg, lens, time_table, W_ihT, W_hhT, b_tot,
               tv_row):
    def body(w_ref, b0_ref, t_ref, tg_ref, len_ref, tt_ref, wih_ref, whh_ref,
             bt_ref, tv_ref, seq_out, fs_out, h_s, c_s, seq_s, fs_s, proj_s):
        step = pl.program_id(0)

        @pl.when(step == 0)
        def _():
            h_s[...] = jnp.zeros_like(h_s)
            c_s[...] = jnp.zeros_like(c_s)
            seq_s[...] = jnp.zeros_like(seq_s)
            fs_s[...] = jnp.zeros_like(fs_s)
            proj_s[...] = lax.dot_general(
                tt_ref[...].astype(jnp.bfloat16), wih_ref[...],
                (((1,), (0,)), ((), ())),
                preferred_element_type=jnp.float32).astype(jnp.bfloat16)

        tvec = t_ref[0]
        onehot = (tvec == lax.broadcasted_iota(jnp.int32, (B2, TIMEN), 1)
                  ).astype(jnp.bfloat16)
        gates = (lax.dot_general(onehot, proj_s[...], (((1,), (0,)), ((), ())),
                                 preferred_element_type=jnp.float32)
                 + lax.dot_general(h_s[...].astype(jnp.bfloat16),
                                   whh_ref[...], (((1,), (0,)), ((), ())),
                                   preferred_element_type=jnp.float32)
                 + bt_ref[...])
        gi = jax.nn.sigmoid(gates[:, 0:H])
        gf = jax.nn.sigmoid(gates[:, H:2 * H])
        gg = jnp.tanh(gates[:, 2 * H:3 * H])
        go = jax.nn.sigmoid(gates[:, 3 * H:4 * H])
        c = gf * c_s[...] + gi * gg
        h = go * jnp.tanh(c)
        h_s[...] = h
        c_s[...] = c

        p1 = jnp.sum(h * tv_ref[...], axis=1, keepdims=True)
        w = w_ref[0, 0]
        b0 = b0_ref[0, 0]
        p2 = w * tg_ref[0]
        f1 = p1 + p2 + b0 + (jnp.exp(p1 + b0) - jnp.exp(p1 + p2 + b0)) / w
        lm1 = len_ref[...] - 1
        fs_s[...] = fs_s[...] + jnp.where(step < lm1, f1, 0.0)
        seq_s[...] = jnp.where(lm1 == step, h, seq_s[...])

        @pl.when(step == S - 1)
        def _():
            seq_out[...] = seq_s[...]
            fs_out[...] = fs_s[...]

    return pl.pallas_call(
        body,
        grid=(S,),
        in_specs=[
            pl.BlockSpec(memory_space=pltpu.SMEM),
            pl.BlockSpec(memory_space=pltpu.SMEM),
            pl.BlockSpec((1, B2, 1), lambda s: (s, 0, 0)),
            pl.BlockSpec((1, B2, 1), lambda s: (s, 0, 0)),
            pl.BlockSpec((B2, 1), lambda s: (0, 0)),
            pl.BlockSpec((TIMEN, ED), lambda s: (0, 0)),
            pl.BlockSpec((ED, 4 * H), lambda s: (0, 0)),
            pl.BlockSpec((H, 4 * H), lambda s: (0, 0)),
            pl.BlockSpec((1, 4 * H), lambda s: (0, 0)),
            pl.BlockSpec((1, H), lambda s: (0, 0)),
        ],
        out_specs=[
            pl.BlockSpec((B2, H), lambda s: (0, 0)),
            pl.BlockSpec((B2, 1), lambda s: (0, 0)),
        ],
        out_shape=[
            jax.ShapeDtypeStruct((B2, H), jnp.float32),
            jax.ShapeDtypeStruct((B2, 1), jnp.float32),
        ],
        scratch_shapes=[
            pltpu.VMEM((B2, H), jnp.float32),
            pltpu.VMEM((B2, H), jnp.float32),
            pltpu.VMEM((B2, H), jnp.float32),
            pltpu.VMEM((B2, 1), jnp.float32),
            pltpu.VMEM((TIMEN, 4 * H), jnp.bfloat16),
        ],
    )(w_s, b_s, times, tg, lens, time_table, W_ihT, W_hhT, b_tot, tv_row)


def _final(hang, lie, seq, u_emb, fsum, lens, w_hang, w_lie, w_time, w_u,
           fcb):
    def body(hang_ref, lie_ref, seq_ref, u_ref, fs_ref, len_ref, wh_ref,
             wl_ref, wt_ref, wu_ref, fb_ref, out_ref, tl_ref):
        th = jnp.tanh(seq_ref[0:B, :] * seq_ref[B:B2, :])
        tu = jnp.tanh(u_ref[0:B, :] * u_ref[B:B2, :])

        def dn(a, w):
            return lax.dot_general(a, w, (((1,), (0,)), ((), ())),
                                   preferred_element_type=jnp.float32)

        out_ref[...] = (dn(hang_ref[...], wh_ref[...])
                        + dn(lie_ref[...], wl_ref[...])
                        + dn(th, wt_ref[...])
                        + dn(tu, wu_ref[...])
                        + fb_ref[...])
        r = fs_ref[...] / (len_ref[...] - 1).astype(jnp.float32)
        tl_ref[...] = (-jnp.sum(r) / B).reshape(1, 1)

    return pl.pallas_call(
        body,
        out_shape=[
            jax.ShapeDtypeStruct((B, 2), jnp.float32),
            jax.ShapeDtypeStruct((1, 1), jnp.float32),
        ],
    )(hang, lie, seq, u_emb, fsum, lens, w_hang, w_lie, w_time, w_u, fcb)


def kernel(u1, u2, length_1, length_2, loc_1, loc_2, time_1, time_2,
           time_gap_1, time_gap_2, loc_table, time_table, W_ih, W_hh, b_ih,
           b_hh, time_v, time_w, time_b, gat_embedding, fc_W, fc_b):
    loc_idx = jnp.concatenate([loc_1.reshape(-1),
                               loc_2.reshape(-1)]).astype(jnp.int32)
    u_idx = jnp.concatenate([u1, u2]).astype(jnp.int32)
    emb1, emb2, emb_u = _sc_gather_loc_user(loc_idx, u_idx, loc_table,
                                            gat_embedding)

    l1 = length_1.astype(jnp.int32)
    l2 = length_2.astype(jnp.int32)
    hang, lie = _cos_topmax(l1, l2, emb1, emb2)

    times = (jnp.concatenate([time_1, time_2], axis=0).astype(jnp.int32)
             .T.reshape(S, B2, 1))
    zero_col = jnp.zeros((B, 1), time_gap_1.dtype)
    tg1 = jnp.concatenate([time_gap_1[:, 1:], zero_col], axis=1)
    tg2 = jnp.concatenate([time_gap_2[:, 1:], zero_col], axis=1)
    tg = jnp.concatenate([tg1, tg2], axis=0).T.reshape(S, B2, 1)
    lens = jnp.concatenate([l1, l2]).reshape(B2, 1)
    seq, fsum = _lstm_loss(time_w, time_b, times, tg, lens, time_table,
                           W_ih.T.astype(jnp.bfloat16),
                           W_hh.T.astype(jnp.bfloat16),
                           (b_ih + b_hh).reshape(1, 4 * H),
                           time_v.reshape(1, H))

    outs, tl = _final(hang, lie, seq, emb_u, fsum, lens, fc_W[0:S],
                      fc_W[S:2 * S], fc_W[2 * S:2 * S + H],
                      fc_W[2 * S + H:], fc_b.reshape(1, 2))
    return (outs, tl.reshape(()))


# speedup vs baseline: 8.8811x; 1.1008x over previous
import functools

import jax
import jax.numpy as jnp
from jax import lax
from jax.experimental import pallas as pl
from jax.experimental.pallas import tpu as pltpu
from jax.experimental.pallas import tpu_sc as plsc

B = 256
S = 200
ED = 64
H = 128
B2 = 2 * B
TIMEN = 168

NW = 32
HALF_PER_W = B * S // NW
CHUNK = 800
U_PER_W = B2 // NW


def _sc_gather_loc_user(loc_idx, u_idx, loc_table, gat_embedding):
    @functools.partial(
        pl.kernel,
        out_type=[
            jax.ShapeDtypeStruct((B * S, ED), jnp.float32),
            jax.ShapeDtypeStruct((B * S, ED), jnp.float32),
            jax.ShapeDtypeStruct((B2, ED), jnp.float32),
        ],
        mesh=plsc.VectorSubcoreMesh(core_axis_name="c", subcore_axis_name="s"),
        scratch_types=[
            [pltpu.VMEM((CHUNK,), jnp.int32)] * 2,
            [pltpu.VMEM((CHUNK, ED), jnp.float32)] * 2,
            pltpu.VMEM((U_PER_W,), jnp.int32),
            pltpu.VMEM((U_PER_W, ED), jnp.float32),
            [pltpu.SemaphoreType.DMA] * 2,
            [pltpu.SemaphoreType.DMA] * 2,
            pltpu.SemaphoreType.DMA,
        ],
        compiler_params=pltpu.CompilerParams(use_tc_tiling_on_sc=False),
    )
    def gather_kernel(loc_idx_h, u_idx_h, loc_t_h, gat_h, e1_out, e2_out,
                      u_out, idx_v, rows_v, uidx_v, urows_v, gsems, wsems,
                      usem):
        wid = lax.axis_index("s") * 2 + lax.axis_index("c")
        ubase = wid * U_PER_W
        pltpu.sync_copy(u_idx_h.at[pl.ds(ubase, U_PER_W)], uidx_v)
        uh = pltpu.async_copy(gat_h.at[uidx_v], urows_v, usem)

        nch = HALF_PER_W // CHUNK
        work = ([(0, e1_out, j) for j in range(nch)]
                + [(B * S, e2_out, j) for j in range(nch)])
        gh = [None, None]
        wh = [None, None]
        pend = [None, None]
        for j in range(len(work) + 1):
            if j < len(work):
                bb = j % 2
                if wh[bb] is not None:
                    wh[bb].wait()
                off, out_h, cj = work[j]
                base = wid * HALF_PER_W + cj * CHUNK
                pltpu.sync_copy(loc_idx_h.at[pl.ds(off + base, CHUNK)],
                                idx_v[bb])
                gh[bb] = pltpu.async_copy(loc_t_h.at[idx_v[bb]], rows_v[bb],
                                          gsems[bb])
                pend[bb] = (out_h, base)
            if j >= 1:
                pb = (j - 1) % 2
                gh[pb].wait()
                out_h, base = pend[pb]
                wh[pb] = pltpu.async_copy(rows_v[pb],
                                          out_h.at[pl.ds(base, CHUNK)],
                                          wsems[pb])
        for h in wh:
            if h is not None:
                h.wait()
        uh.wait()
        pltpu.sync_copy(urows_v, u_out.at[pl.ds(ubase, U_PER_W)])

    return gather_kernel(loc_idx, u_idx, loc_table, gat_embedding)


COS_BATCH = 8


def _cos_topmax(length_1, length_2, emb1, emb2):
    def body(l1_ref, l2_ref, e1_ref, e2_ref, hang_ref, lie_ref):
        b0 = pl.program_id(0) * COS_BATCH
        neg = jnp.float32(-jnp.inf)
        hang_cols = []
        lie_rows = []
        for i in range(COS_BATCH):
            e1 = e1_ref[pl.ds(i * S, S), :]
            e2 = e2_ref[pl.ds(i * S, S), :]
            r1 = lax.rsqrt(jnp.sum(e1 * e1, axis=1, keepdims=True))
            r2 = lax.rsqrt(jnp.sum(e2 * e2, axis=1, keepdims=True))
            cos = lax.dot_general(e1 * r1, e2 * r2, (((1,), (1,)), ((), ())),
                                  preferred_element_type=jnp.float32)
            l1 = l1_ref[b0 + i]
            l2 = l2_ref[b0 + i]
            row_id = lax.broadcasted_iota(jnp.int32, (S, S), 0)
            col_id = lax.broadcasted_iota(jnp.int32, (S, S), 1)
            hang_full = jnp.max(jnp.where(col_id < l2, cos, neg), axis=1,
                                keepdims=True)
            rmask = lax.broadcasted_iota(jnp.int32, (S, 1), 0) < l1
            hang_cols.append(jnp.where(rmask, hang_full, 0.0))
            lie_full = jnp.max(jnp.where(row_id < l1, cos, neg), axis=0,
                               keepdims=True)
            cmask = lax.broadcasted_iota(jnp.int32, (1, S), 1) < l2
            lie_rows.append(jnp.where(cmask, lie_full, 0.0))
        hang_ref[...] = jnp.concatenate(hang_cols, axis=1).T
        lie_ref[...] = jnp.concatenate(lie_rows, axis=0)

    return pl.pallas_call(
        body,
        grid=(B // COS_BATCH,),
        in_specs=[
            pl.BlockSpec(memory_space=pltpu.SMEM),
            pl.BlockSpec(memory_space=pltpu.SMEM),
            pl.BlockSpec((COS_BATCH * S, ED), lambda b: (b, 0)),
            pl.BlockSpec((COS_BATCH * S, ED), lambda b: (b, 0)),
        ],
        out_specs=[
            pl.BlockSpec((COS_BATCH, S), lambda b: (b, 0)),
            pl.BlockSpec((COS_BATCH, S), lambda b: (b, 0)),
        ],
        out_shape=[
            jax.ShapeDtypeStruct((B, S), jnp.float32),
            jax.ShapeDtypeStruct((B, S), jnp.float32),
        ],
    )(length_1, length_2, emb1, emb2)


def _lstm_loss(w_s, b_s, tgt, lens, time_table, W_ihT, W_hhT, b_tot, tv_row):
    def body(w_ref, b0_ref, tgt_ref, len_ref, tt_ref, wih_ref, whh_ref,
             bt_ref, tv_ref, seq_out, fs_out, h_s, c_s, seq_s, fs_s, proj_s):
        step = pl.program_id(0)

        @pl.when(step == 0)
        def _():
            h_s[...] = jnp.zeros_like(h_s)
            c_s[...] = jnp.zeros_like(c_s)
            seq_s[...] = jnp.zeros_like(seq_s)
            fs_s[...] = jnp.zeros_like(fs_s)
            proj_s[...] = lax.dot_general(
                tt_ref[...].astype(jnp.bfloat16), wih_ref[...],
                (((1,), (0,)), ((), ())),
                preferred_element_type=jnp.float32).astype(jnp.bfloat16)

        ri = lax.broadcasted_iota(jnp.int32, (2 * S, 2), 0)
        ci = lax.broadcasted_iota(jnp.int32, (2 * S, 2), 1)
        sel = (ri == step + S * ci).astype(jnp.float32)
        tc2 = lax.dot_general(tgt_ref[...], sel, (((1,), (0,)), ((), ())),
                              preferred_element_type=jnp.float32)
        tgv = tc2[:, 0:1]
        tvec = tc2[:, 1:2].astype(jnp.int32)
        onehot = (tvec == lax.broadcasted_iota(jnp.int32, (B2, TIMEN), 1)
                  ).astype(jnp.bfloat16)
        gates = (lax.dot_general(onehot, proj_s[...], (((1,), (0,)), ((), ())),
                                 preferred_element_type=jnp.float32)
                 + lax.dot_general(h_s[...].astype(jnp.bfloat16),
                                   whh_ref[...], (((1,), (0,)), ((), ())),
                                   preferred_element_type=jnp.float32)
                 + bt_ref[...])
        gi = jax.nn.sigmoid(gates[:, 0:H])
        gf = jax.nn.sigmoid(gates[:, H:2 * H])
        gg = jnp.tanh(gates[:, 2 * H:3 * H])
        go = jax.nn.sigmoid(gates[:, 3 * H:4 * H])
        c = gf * c_s[...] + gi * gg
        h = go * jnp.tanh(c)
        h_s[...] = h
        c_s[...] = c

        p1 = jnp.sum(h * tv_ref[...], axis=1, keepdims=True)
        w = w_ref[0, 0]
        b0 = b0_ref[0, 0]
        p2 = w * tgv
        f1 = p1 + p2 + b0 + (jnp.exp(p1 + b0) - jnp.exp(p1 + p2 + b0)) / w
        lm1 = len_ref[...] - 1
        fs_s[...] = fs_s[...] + jnp.where(step < lm1, f1, 0.0)
        seq_s[...] = jnp.where(lm1 == step, h, seq_s[...])

        @pl.when(step == S - 1)
        def _():
            seq_out[...] = seq_s[...]
            fs_out[...] = fs_s[...]

    return pl.pallas_call(
        body,
        grid=(S,),
        in_specs=[
            pl.BlockSpec(memory_space=pltpu.SMEM),
            pl.BlockSpec(memory_space=pltpu.SMEM),
            pl.BlockSpec((B2, 2 * S), lambda s: (0, 0)),
            pl.BlockSpec((B2, 1), lambda s: (0, 0)),
            pl.BlockSpec((TIMEN, ED), lambda s: (0, 0)),
            pl.BlockSpec((ED, 4 * H), lambda s: (0, 0)),
            pl.BlockSpec((H, 4 * H), lambda s: (0, 0)),
            pl.BlockSpec((1, 4 * H), lambda s: (0, 0)),
            pl.BlockSpec((1, H), lambda s: (0, 0)),
        ],
        out_specs=[
            pl.BlockSpec((B2, H), lambda s: (0, 0)),
            pl.BlockSpec((B2, 1), lambda s: (0, 0)),
        ],
        out_shape=[
            jax.ShapeDtypeStruct((B2, H), jnp.float32),
            jax.ShapeDtypeStruct((B2, 1), jnp.float32),
        ],
        scratch_shapes=[
            pltpu.VMEM((B2, H), jnp.float32),
            pltpu.VMEM((B2, H), jnp.float32),
            pltpu.VMEM((B2, H), jnp.float32),
            pltpu.VMEM((B2, 1), jnp.float32),
            pltpu.VMEM((TIMEN, 4 * H), jnp.bfloat16),
        ],
    )(w_s, b_s, tgt, lens, time_table, W_ihT, W_hhT, b_tot, tv_row)


def _final(hang, lie, seq, u_emb, fsum, lens, w_hang, w_lie, w_time, w_u,
           fcb):
    def body(hang_ref, lie_ref, seq_ref, u_ref, fs_ref, len_ref, wh_ref,
             wl_ref, wt_ref, wu_ref, fb_ref, out_ref, tl_ref):
        th = jnp.tanh(seq_ref[0:B, :] * seq_ref[B:B2, :])
        tu = jnp.tanh(u_ref[0:B, :] * u_ref[B:B2, :])

        def dn(a, w):
            return lax.dot_general(a, w, (((1,), (0,)), ((), ())),
                                   preferred_element_type=jnp.float32)

        out_ref[...] = (dn(hang_ref[...], wh_ref[...])
                        + dn(lie_ref[...], wl_ref[...])
                        + dn(th, wt_ref[...])
                        + dn(tu, wu_ref[...])
                        + fb_ref[...])
        r = fs_ref[...] / (len_ref[...] - 1).astype(jnp.float32)
        tl_ref[...] = (-jnp.sum(r) / B).reshape(1, 1)

    return pl.pallas_call(
        body,
        out_shape=[
            jax.ShapeDtypeStruct((B, 2), jnp.float32),
            jax.ShapeDtypeStruct((1, 1), jnp.float32),
        ],
    )(hang, lie, seq, u_emb, fsum, lens, w_hang, w_lie, w_time, w_u, fcb)


def kernel(u1, u2, length_1, length_2, loc_1, loc_2, time_1, time_2,
           time_gap_1, time_gap_2, loc_table, time_table, W_ih, W_hh, b_ih,
           b_hh, time_v, time_w, time_b, gat_embedding, fc_W, fc_b):
    loc_idx = jnp.concatenate([loc_1.reshape(-1),
                               loc_2.reshape(-1)]).astype(jnp.int32)
    u_idx = jnp.concatenate([u1, u2]).astype(jnp.int32)
    emb1, emb2, emb_u = _sc_gather_loc_user(loc_idx, u_idx, loc_table,
                                            gat_embedding)

    l1 = length_1.astype(jnp.int32)
    l2 = length_2.astype(jnp.int32)
    hang, lie = _cos_topmax(l1, l2, emb1, emb2)

    times_f = jnp.concatenate([time_1, time_2], axis=0).astype(jnp.float32)
    zero_col = jnp.zeros((B, 1), time_gap_1.dtype)
    tg1 = jnp.concatenate([time_gap_1[:, 1:], zero_col], axis=1)
    tg2 = jnp.concatenate([time_gap_2[:, 1:], zero_col], axis=1)
    tgt = jnp.concatenate([jnp.concatenate([tg1, tg2], axis=0), times_f],
                          axis=1)
    lens = jnp.concatenate([l1, l2]).reshape(B2, 1)
    seq, fsum = _lstm_loss(time_w, time_b, tgt, lens, time_table,
                           W_ih.T.astype(jnp.bfloat16),
                           W_hh.T.astype(jnp.bfloat16),
                           (b_ih + b_hh).reshape(1, 4 * H),
                           time_v.reshape(1, H))

    outs, tl = _final(hang, lie, seq, emb_u, fsum, lens, fc_W[0:S],
                      fc_W[S:2 * S], fc_W[2 * S:2 * S + H],
                      fc_W[2 * S + H:], fc_b.reshape(1, 2))
    return (outs, tl.reshape(()))


# speedup vs baseline: 10.7234x; 1.2074x over previous
import functools

import jax
import jax.numpy as jnp
from jax import lax
from jax.experimental import pallas as pl
from jax.experimental.pallas import tpu as pltpu
from jax.experimental.pallas import tpu_sc as plsc

B = 256
S = 200
ED = 64
H = 128
B2 = 2 * B
TIMEN = 168

NW = 32
HALF_PER_W = B * S // NW
CHUNK = 800
U_PER_W = B2 // NW


def _sc_gather_loc_user(loc_idx, u_idx, loc_table, gat_embedding):
    @functools.partial(
        pl.kernel,
        out_type=[
            jax.ShapeDtypeStruct((B * S, ED), jnp.float32),
            jax.ShapeDtypeStruct((B * S, ED), jnp.float32),
            jax.ShapeDtypeStruct((B2, ED), jnp.float32),
        ],
        mesh=plsc.VectorSubcoreMesh(core_axis_name="c", subcore_axis_name="s"),
        scratch_types=[
            [pltpu.VMEM((CHUNK,), jnp.int32)] * 2,
            [pltpu.VMEM((CHUNK, ED), jnp.float32)] * 2,
            pltpu.VMEM((U_PER_W,), jnp.int32),
            pltpu.VMEM((U_PER_W, ED), jnp.float32),
            [pltpu.SemaphoreType.DMA] * 2,
            [pltpu.SemaphoreType.DMA] * 2,
            pltpu.SemaphoreType.DMA,
        ],
        compiler_params=pltpu.CompilerParams(use_tc_tiling_on_sc=False),
    )
    def gather_kernel(loc_idx_h, u_idx_h, loc_t_h, gat_h, e1_out, e2_out,
                      u_out, idx_v, rows_v, uidx_v, urows_v, gsems, wsems,
                      usem):
        wid = lax.axis_index("s") * 2 + lax.axis_index("c")
        ubase = wid * U_PER_W
        pltpu.sync_copy(u_idx_h.at[pl.ds(ubase, U_PER_W)], uidx_v)
        uh = pltpu.async_copy(gat_h.at[uidx_v], urows_v, usem)

        nch = HALF_PER_W // CHUNK
        work = ([(0, e1_out, j) for j in range(nch)]
                + [(B * S, e2_out, j) for j in range(nch)])
        gh = [None, None]
        wh = [None, None]
        pend = [None, None]
        for j in range(len(work) + 1):
            if j < len(work):
                bb = j % 2
                if wh[bb] is not None:
                    wh[bb].wait()
                off, out_h, cj = work[j]
                base = wid * HALF_PER_W + cj * CHUNK
                pltpu.sync_copy(loc_idx_h.at[pl.ds(off + base, CHUNK)],
                                idx_v[bb])
                gh[bb] = pltpu.async_copy(loc_t_h.at[idx_v[bb]], rows_v[bb],
                                          gsems[bb])
                pend[bb] = (out_h, base)
            if j >= 1:
                pb = (j - 1) % 2
                gh[pb].wait()
                out_h, base = pend[pb]
                wh[pb] = pltpu.async_copy(rows_v[pb],
                                          out_h.at[pl.ds(base, CHUNK)],
                                          wsems[pb])
        for h in wh:
            if h is not None:
                h.wait()
        uh.wait()
        pltpu.sync_copy(urows_v, u_out.at[pl.ds(ubase, U_PER_W)])

    return gather_kernel(loc_idx, u_idx, loc_table, gat_embedding)


COS_BATCH = 8


def _cos_topmax(length_1, length_2, emb1, emb2):
    def body(l1_ref, l2_ref, e1_ref, e2_ref, hang_ref, lie_ref):
        b0 = pl.program_id(0) * COS_BATCH
        neg = jnp.float32(-jnp.inf)
        hang_cols = []
        lie_rows = []
        for i in range(COS_BATCH):
            e1 = e1_ref[pl.ds(i * S, S), :]
            e2 = e2_ref[pl.ds(i * S, S), :]
            r1 = lax.rsqrt(jnp.sum(e1 * e1, axis=1, keepdims=True))
            r2 = lax.rsqrt(jnp.sum(e2 * e2, axis=1, keepdims=True))
            cos = lax.dot_general(e1 * r1, e2 * r2, (((1,), (1,)), ((), ())),
                                  preferred_element_type=jnp.float32)
            l1 = l1_ref[b0 + i]
            l2 = l2_ref[b0 + i]
            row_id = lax.broadcasted_iota(jnp.int32, (S, S), 0)
            col_id = lax.broadcasted_iota(jnp.int32, (S, S), 1)
            hang_full = jnp.max(jnp.where(col_id < l2, cos, neg), axis=1,
                                keepdims=True)
            rmask = lax.broadcasted_iota(jnp.int32, (S, 1), 0) < l1
            hang_cols.append(jnp.where(rmask, hang_full, 0.0))
            lie_full = jnp.max(jnp.where(row_id < l1, cos, neg), axis=0,
                               keepdims=True)
            cmask = lax.broadcasted_iota(jnp.int32, (1, S), 1) < l2
            lie_rows.append(jnp.where(cmask, lie_full, 0.0))
        hang_ref[...] = jnp.concatenate(hang_cols, axis=1).T
        lie_ref[...] = jnp.concatenate(lie_rows, axis=0)

    return pl.pallas_call(
        body,
        grid=(B // COS_BATCH,),
        in_specs=[
            pl.BlockSpec(memory_space=pltpu.SMEM),
            pl.BlockSpec(memory_space=pltpu.SMEM),
            pl.BlockSpec((COS_BATCH * S, ED), lambda b: (b, 0)),
            pl.BlockSpec((COS_BATCH * S, ED), lambda b: (b, 0)),
        ],
        out_specs=[
            pl.BlockSpec((COS_BATCH, S), lambda b: (b, 0)),
            pl.BlockSpec((COS_BATCH, S), lambda b: (b, 0)),
        ],
        out_shape=[
            jax.ShapeDtypeStruct((B, S), jnp.float32),
            jax.ShapeDtypeStruct((B, S), jnp.float32),
        ],
    )(length_1, length_2, emb1, emb2)


LSTM_UNROLL = 10


def _lstm_loss(w_s, b_s, tgt, lens, time_table, W_ihT, W_hhT, b_tot, tv_row):
    def body(w_ref, b0_ref, tgt_ref, len_ref, tt_ref, wih_ref, whh_ref,
             bt_ref, tv_ref, seq_out, fs_out, h_s, c_s, seq_s, fs_s, proj_s):
        pid = pl.program_id(0)

        @pl.when(pid == 0)
        def _():
            h_s[...] = jnp.zeros_like(h_s)
            c_s[...] = jnp.zeros_like(c_s)
            seq_s[...] = jnp.zeros_like(seq_s)
            fs_s[...] = jnp.zeros_like(fs_s)
            proj_s[...] = lax.dot_general(
                tt_ref[...].astype(jnp.bfloat16), wih_ref[...],
                (((1,), (0,)), ((), ())),
                preferred_element_type=jnp.float32).astype(jnp.bfloat16)

        w = w_ref[0, 0]
        b0 = b0_ref[0, 0]
        lm1 = len_ref[...] - 1
        hb = h_s[...]
        c = c_s[...]
        seq = seq_s[...]
        fs = fs_s[...]
        for u in range(LSTM_UNROLL):
            step = pid * LSTM_UNROLL + u
            ri = lax.broadcasted_iota(jnp.int32, (2 * S, 2), 0)
            ci = lax.broadcasted_iota(jnp.int32, (2 * S, 2), 1)
            sel = (ri == step + S * ci).astype(jnp.bfloat16)
            tc2 = lax.dot_general(tgt_ref[...], sel, (((1,), (0,)), ((), ())),
                                  preferred_element_type=jnp.float32)
            tgv = tc2[:, 0:1]
            onehot = (tc2[:, 1:2].astype(jnp.int32)
                      == lax.broadcasted_iota(jnp.int32, (B2, TIMEN), 1)
                      ).astype(jnp.bfloat16)
            gates = (lax.dot_general(onehot, proj_s[...],
                                     (((1,), (0,)), ((), ())),
                                     preferred_element_type=jnp.float32)
                     + lax.dot_general(hb, whh_ref[...],
                                       (((1,), (0,)), ((), ())),
                                       preferred_element_type=jnp.float32)
                     + bt_ref[...])
            gi = jax.nn.sigmoid(gates[:, 0:H])
            gf = jax.nn.sigmoid(gates[:, H:2 * H])
            gg = jnp.tanh(gates[:, 2 * H:3 * H])
            go = jax.nn.sigmoid(gates[:, 3 * H:4 * H])
            c = gf * c + gi * gg
            h = go * jnp.tanh(c)
            hb = h.astype(jnp.bfloat16)

            p1 = jnp.sum(h * tv_ref[...], axis=1, keepdims=True)
            p2 = w * tgv
            f1 = (p1 + p2 + b0
                  + (jnp.exp(p1 + b0) - jnp.exp(p1 + p2 + b0)) / w)
            fs = fs + jnp.where(step < lm1, f1, 0.0)
            seq = jnp.where(lm1 == step, h, seq)
        h_s[...] = hb
        c_s[...] = c
        seq_s[...] = seq
        fs_s[...] = fs

        @pl.when(pid == S // LSTM_UNROLL - 1)
        def _():
            seq_out[...] = seq
            fs_out[...] = fs

    return pl.pallas_call(
        body,
        grid=(S // LSTM_UNROLL,),
        in_specs=[
            pl.BlockSpec(memory_space=pltpu.SMEM),
            pl.BlockSpec(memory_space=pltpu.SMEM),
            pl.BlockSpec((B2, 2 * S), lambda s: (0, 0)),
            pl.BlockSpec((B2, 1), lambda s: (0, 0)),
            pl.BlockSpec((TIMEN, ED), lambda s: (0, 0)),
            pl.BlockSpec((ED, 4 * H), lambda s: (0, 0)),
            pl.BlockSpec((H, 4 * H), lambda s: (0, 0)),
            pl.BlockSpec((1, 4 * H), lambda s: (0, 0)),
            pl.BlockSpec((1, H), lambda s: (0, 0)),
        ],
        out_specs=[
            pl.BlockSpec((B2, H), lambda s: (0, 0)),
            pl.BlockSpec((B2, 1), lambda s: (0, 0)),
        ],
        out_shape=[
            jax.ShapeDtypeStruct((B2, H), jnp.float32),
            jax.ShapeDtypeStruct((B2, 1), jnp.float32),
        ],
        scratch_shapes=[
            pltpu.VMEM((B2, H), jnp.bfloat16),
            pltpu.VMEM((B2, H), jnp.float32),
            pltpu.VMEM((B2, H), jnp.float32),
            pltpu.VMEM((B2, 1), jnp.float32),
            pltpu.VMEM((TIMEN, 4 * H), jnp.bfloat16),
        ],
    )(w_s, b_s, tgt, lens, time_table, W_ihT, W_hhT, b_tot, tv_row)


def _final(hang, lie, seq, u_emb, fsum, lens, w_hang, w_lie, w_time, w_u,
           fcb):
    def body(hang_ref, lie_ref, seq_ref, u_ref, fs_ref, len_ref, wh_ref,
             wl_ref, wt_ref, wu_ref, fb_ref, out_ref, tl_ref):
        th = jnp.tanh(seq_ref[0:B, :] * seq_ref[B:B2, :])
        tu = jnp.tanh(u_ref[0:B, :] * u_ref[B:B2, :])

        def dn(a, w):
            return lax.dot_general(a, w, (((1,), (0,)), ((), ())),
                                   preferred_element_type=jnp.float32)

        out_ref[...] = (dn(hang_ref[...], wh_ref[...])
                        + dn(lie_ref[...], wl_ref[...])
                        + dn(th, wt_ref[...])
                        + dn(tu, wu_ref[...])
                        + fb_ref[...])
        r = fs_ref[...] / (len_ref[...] - 1).astype(jnp.float32)
        tl_ref[...] = (-jnp.sum(r) / B).reshape(1, 1)

    return pl.pallas_call(
        body,
        out_shape=[
            jax.ShapeDtypeStruct((B, 2), jnp.float32),
            jax.ShapeDtypeStruct((1, 1), jnp.float32),
        ],
    )(hang, lie, seq, u_emb, fsum, lens, w_hang, w_lie, w_time, w_u, fcb)


def kernel(u1, u2, length_1, length_2, loc_1, loc_2, time_1, time_2,
           time_gap_1, time_gap_2, loc_table, time_table, W_ih, W_hh, b_ih,
           b_hh, time_v, time_w, time_b, gat_embedding, fc_W, fc_b):
    loc_idx = jnp.concatenate([loc_1.reshape(-1),
                               loc_2.reshape(-1)]).astype(jnp.int32)
    u_idx = jnp.concatenate([u1, u2]).astype(jnp.int32)
    emb1, emb2, emb_u = _sc_gather_loc_user(loc_idx, u_idx, loc_table,
                                            gat_embedding)

    l1 = length_1.astype(jnp.int32)
    l2 = length_2.astype(jnp.int32)
    hang, lie = _cos_topmax(l1, l2, emb1, emb2)

    times_f = jnp.concatenate([time_1, time_2], axis=0).astype(jnp.float32)
    zero_col = jnp.zeros((B, 1), time_gap_1.dtype)
    tg1 = jnp.concatenate([time_gap_1[:, 1:], zero_col], axis=1)
    tg2 = jnp.concatenate([time_gap_2[:, 1:], zero_col], axis=1)
    tgt = jnp.concatenate([jnp.concatenate([tg1, tg2], axis=0), times_f],
                          axis=1).astype(jnp.bfloat16)
    lens = jnp.concatenate([l1, l2]).reshape(B2, 1)
    seq, fsum = _lstm_loss(time_w, time_b, tgt, lens, time_table,
                           W_ih.T.astype(jnp.bfloat16),
                           W_hh.T.astype(jnp.bfloat16),
                           (b_ih + b_hh).reshape(1, 4 * H),
                           time_v.reshape(1, H))

    outs, tl = _final(hang, lie, seq, emb_u, fsum, lens, fc_W[0:S],
                      fc_W[S:2 * S], fc_W[2 * S:2 * S + H],
                      fc_W[2 * S + H:], fc_b.reshape(1, 2))
    return (outs, tl.reshape(()))


# speedup vs baseline: 11.3263x; 1.0562x over previous
import functools

import jax
import jax.numpy as jnp
from jax import lax
from jax.experimental import pallas as pl
from jax.experimental.pallas import tpu as pltpu
from jax.experimental.pallas import tpu_sc as plsc

B = 256
S = 200
ED = 64
H = 128
B2 = 2 * B
TIMEN = 168

NW = 32
HALF_PER_W = B * S // NW
CHUNK = 800
U_PER_W = B2 // NW


def _sc_gather_loc_user(loc_idx, u_idx, loc_table, gat_embedding):
    @functools.partial(
        pl.kernel,
        out_type=[
            jax.ShapeDtypeStruct((B * S, 2 * ED), jnp.float32),
            jax.ShapeDtypeStruct((B, 2 * ED), jnp.float32),
        ],
        mesh=plsc.VectorSubcoreMesh(core_axis_name="c", subcore_axis_name="s"),
        scratch_types=[
            [pltpu.VMEM((CHUNK,), jnp.int32)] * 2,
            [pltpu.VMEM((CHUNK, ED), jnp.float32)] * 2,
            pltpu.VMEM((U_PER_W,), jnp.int32),
            pltpu.VMEM((U_PER_W, ED), jnp.float32),
            [pltpu.SemaphoreType.DMA] * 2,
            [pltpu.SemaphoreType.DMA] * 2,
            pltpu.SemaphoreType.DMA,
        ],
        compiler_params=pltpu.CompilerParams(use_tc_tiling_on_sc=False),
    )
    def gather_kernel(loc_idx_h, u_idx_h, loc_t_h, gat_h, e_out,
                      u_out, idx_v, rows_v, uidx_v, urows_v, gsems, wsems,
                      usem):
        wid = lax.axis_index("s") * 2 + lax.axis_index("c")
        ubase = wid * U_PER_W
        pltpu.sync_copy(u_idx_h.at[pl.ds(ubase, U_PER_W)], uidx_v)
        uh = pltpu.async_copy(gat_h.at[uidx_v], urows_v, usem)

        nch = HALF_PER_W // CHUNK
        work = [(B * S * h, h * ED, j)
                for h in (0, 1)
                for j in range(nch)]
        gh = [None, None]
        wh = [None, None]
        pend = [None, None]
        for j in range(len(work) + 1):
            if j < len(work):
                bb = j % 2
                if wh[bb] is not None:
                    wh[bb].wait()
                off, col, cj = work[j]
                base = wid * HALF_PER_W + cj * CHUNK
                pltpu.sync_copy(loc_idx_h.at[pl.ds(off + base, CHUNK)],
                                idx_v[bb])
                gh[bb] = pltpu.async_copy(loc_t_h.at[idx_v[bb]], rows_v[bb],
                                          gsems[bb])
                pend[bb] = (col, base)
            if j >= 1:
                pb = (j - 1) % 2
                gh[pb].wait()
                col, base = pend[pb]
                wh[pb] = pltpu.async_copy(
                    rows_v[pb],
                    e_out.at[pl.ds(base, CHUNK), pl.ds(col, ED)],
                    wsems[pb])
        for h in wh:
            if h is not None:
                h.wait()
        uh.wait()
        is_u2 = (wid >= NW // 2).astype(jnp.int32)
        urow = ubase - B * is_u2
        ucol = ED * is_u2
        pltpu.sync_copy(urows_v,
                        u_out.at[pl.ds(urow, U_PER_W), pl.ds(ucol, ED)])

    return gather_kernel(loc_idx, u_idx, loc_table, gat_embedding)


COS_BATCH = 8


def _cos_topmax(length_1, length_2, emb):
    def body(l1_ref, l2_ref, e_ref, hang_ref, lie_ref):
        b0 = pl.program_id(0) * COS_BATCH
        neg = jnp.float32(-jnp.inf)
        x = e_ref[...]
        e1a = x[:, 0:ED]
        e2a = x[:, ED:2 * ED]
        r1a = lax.rsqrt(jnp.sum(e1a * e1a, axis=1, keepdims=True))
        r2a = lax.rsqrt(jnp.sum(e2a * e2a, axis=1, keepdims=True))
        n1a = e1a * r1a
        n2a = e2a * r2a
        hang_cols = []
        lie_rows = []
        for i in range(COS_BATCH):
            cos = lax.dot_general(n1a[i * S:(i + 1) * S, :],
                                  n2a[i * S:(i + 1) * S, :],
                                  (((1,), (1,)), ((), ())),
                                  preferred_element_type=jnp.float32)
            l1 = l1_ref[b0 + i]
            l2 = l2_ref[b0 + i]
            row_id = lax.broadcasted_iota(jnp.int32, (S, S), 0)
            col_id = lax.broadcasted_iota(jnp.int32, (S, S), 1)
            hang_full = jnp.max(jnp.where(col_id < l2, cos, neg), axis=1,
                                keepdims=True)
            rmask = lax.broadcasted_iota(jnp.int32, (S, 1), 0) < l1
            hang_cols.append(jnp.where(rmask, hang_full, 0.0))
            lie_full = jnp.max(jnp.where(row_id < l1, cos, neg), axis=0,
                               keepdims=True)
            cmask = lax.broadcasted_iota(jnp.int32, (1, S), 1) < l2
            lie_rows.append(jnp.where(cmask, lie_full, 0.0))
        hang_ref[...] = jnp.concatenate(hang_cols, axis=1).T
        lie_ref[...] = jnp.concatenate(lie_rows, axis=0)

    return pl.pallas_call(
        body,
        grid=(B // COS_BATCH,),
        in_specs=[
            pl.BlockSpec(memory_space=pltpu.SMEM),
            pl.BlockSpec(memory_space=pltpu.SMEM),
            pl.BlockSpec((COS_BATCH * S, 2 * ED), lambda b: (b, 0)),
        ],
        out_specs=[
            pl.BlockSpec((COS_BATCH, S), lambda b: (b, 0)),
            pl.BlockSpec((COS_BATCH, S), lambda b: (b, 0)),
        ],
        out_shape=[
            jax.ShapeDtypeStruct((B, S), jnp.float32),
            jax.ShapeDtypeStruct((B, S), jnp.float32),
        ],
    )(length_1, length_2, emb)


LSTM_UNROLL = 10


def _lstm_loss(w_s, b_s, tgt, lens, time_table, W_ihT, W_hhT, b_tot, tv_row):
    def body(w_ref, b0_ref, tgt_ref, len_ref, tt_ref, wih_ref, whh_ref,
             bt_ref, tv_ref, seq_out, fs_out, h_s, c_s, seq_s, fs_s, proj_s):
        pid = pl.program_id(0)

        @pl.when(pid == 0)
        def _():
            h_s[...] = jnp.zeros_like(h_s)
            c_s[...] = jnp.zeros_like(c_s)
            seq_s[...] = jnp.zeros_like(seq_s)
            fs_s[...] = jnp.zeros_like(fs_s)
            proj_s[...] = lax.dot_general(
                tt_ref[...].astype(jnp.bfloat16), wih_ref[...],
                (((1,), (0,)), ((), ())),
                preferred_element_type=jnp.float32).astype(jnp.bfloat16)

        w = w_ref[0, 0]
        b0 = b0_ref[0, 0]
        lm1 = len_ref[...] - 1
        hb = h_s[...]
        c = c_s[...]
        seq = seq_s[...]
        fs = fs_s[...]
        for u in range(LSTM_UNROLL):
            step = pid * LSTM_UNROLL + u
            ri = lax.broadcasted_iota(jnp.int32, (2 * S, 2), 0)
            ci = lax.broadcasted_iota(jnp.int32, (2 * S, 2), 1)
            sel = (ri == step + S * ci).astype(jnp.bfloat16)
            tc2 = lax.dot_general(tgt_ref[...], sel, (((1,), (0,)), ((), ())),
                                  preferred_element_type=jnp.float32)
            tgv = tc2[:, 0:1]
            onehot = (tc2[:, 1:2].astype(jnp.int32)
                      == lax.broadcasted_iota(jnp.int32, (B2, TIMEN), 1)
                      ).astype(jnp.bfloat16)
            gates = (lax.dot_general(onehot, proj_s[...],
                                     (((1,), (0,)), ((), ())),
                                     preferred_element_type=jnp.float32)
                     + lax.dot_general(hb, whh_ref[...],
                                       (((1,), (0,)), ((), ())),
                                       preferred_element_type=jnp.float32)
                     + bt_ref[...])
            gi = jax.nn.sigmoid(gates[:, 0:H])
            gf = jax.nn.sigmoid(gates[:, H:2 * H])
            gg = jnp.tanh(gates[:, 2 * H:3 * H])
            go = jax.nn.sigmoid(gates[:, 3 * H:4 * H])
            c = gf * c + gi * gg
            h = go * jnp.tanh(c)
            hb = h.astype(jnp.bfloat16)

            p1 = jnp.sum(h * tv_ref[...], axis=1, keepdims=True)
            p2 = w * tgv
            f1 = (p1 + p2 + b0
                  + (jnp.exp(p1 + b0) - jnp.exp(p1 + p2 + b0)) / w)
            fs = fs + jnp.where(step < lm1, f1, 0.0)
            seq = jnp.where(lm1 == step, h, seq)
        h_s[...] = hb
        c_s[...] = c
        seq_s[...] = seq
        fs_s[...] = fs

        @pl.when(pid == S // LSTM_UNROLL - 1)
        def _():
            seq_out[...] = seq
            fs_out[...] = fs

    return pl.pallas_call(
        body,
        grid=(S // LSTM_UNROLL,),
        in_specs=[
            pl.BlockSpec(memory_space=pltpu.SMEM),
            pl.BlockSpec(memory_space=pltpu.SMEM),
            pl.BlockSpec((B2, 2 * S), lambda s: (0, 0)),
            pl.BlockSpec((B2, 1), lambda s: (0, 0)),
            pl.BlockSpec((TIMEN, ED), lambda s: (0, 0)),
            pl.BlockSpec((ED, 4 * H), lambda s: (0, 0)),
            pl.BlockSpec((H, 4 * H), lambda s: (0, 0)),
            pl.BlockSpec((1, 4 * H), lambda s: (0, 0)),
            pl.BlockSpec((1, H), lambda s: (0, 0)),
        ],
        out_specs=[
            pl.BlockSpec((B2, H), lambda s: (0, 0)),
            pl.BlockSpec((B2, 1), lambda s: (0, 0)),
        ],
        out_shape=[
            jax.ShapeDtypeStruct((B2, H), jnp.float32),
            jax.ShapeDtypeStruct((B2, 1), jnp.float32),
        ],
        scratch_shapes=[
            pltpu.VMEM((B2, H), jnp.bfloat16),
            pltpu.VMEM((B2, H), jnp.float32),
            pltpu.VMEM((B2, H), jnp.float32),
            pltpu.VMEM((B2, 1), jnp.float32),
            pltpu.VMEM((TIMEN, 4 * H), jnp.bfloat16),
        ],
    )(w_s, b_s, tgt, lens, time_table, W_ihT, W_hhT, b_tot, tv_row)


def _final(hang, lie, seq, u_emb, fsum, lens, w_hang, w_lie, w_time, w_u,
           fcb):
    def body(hang_ref, lie_ref, seq_ref, u_ref, fs_ref, len_ref, wh_ref,
             wl_ref, wt_ref, wu_ref, fb_ref, out_ref, tl_ref):
        th = jnp.tanh(seq_ref[0:B, :] * seq_ref[B:B2, :])
        tu = jnp.tanh(u_ref[:, 0:ED] * u_ref[:, ED:2 * ED])

        def dn(a, w):
            return lax.dot_general(a, w, (((1,), (0,)), ((), ())),
                                   preferred_element_type=jnp.float32)

        out_ref[...] = (dn(hang_ref[...], wh_ref[...])
                        + dn(lie_ref[...], wl_ref[...])
                        + dn(th, wt_ref[...])
                        + dn(tu, wu_ref[...])
                        + fb_ref[...])
        r = fs_ref[...] / (len_ref[...] - 1).astype(jnp.float32)
        tl_ref[...] = (-jnp.sum(r) / B).reshape(1, 1)

    return pl.pallas_call(
        body,
        out_shape=[
            jax.ShapeDtypeStruct((B, 2), jnp.float32),
            jax.ShapeDtypeStruct((1, 1), jnp.float32),
        ],
    )(hang, lie, seq, u_emb, fsum, lens, w_hang, w_lie, w_time, w_u, fcb)


def kernel(u1, u2, length_1, length_2, loc_1, loc_2, time_1, time_2,
           time_gap_1, time_gap_2, loc_table, time_table, W_ih, W_hh, b_ih,
           b_hh, time_v, time_w, time_b, gat_embedding, fc_W, fc_b):
    loc_idx = jnp.concatenate([loc_1.reshape(-1),
                               loc_2.reshape(-1)]).astype(jnp.int32)
    u_idx = jnp.concatenate([u1, u2]).astype(jnp.int32)
    emb, emb_u = _sc_gather_loc_user(loc_idx, u_idx, loc_table,
                                     gat_embedding)

    l1 = length_1.astype(jnp.int32)
    l2 = length_2.astype(jnp.int32)
    hang, lie = _cos_topmax(l1, l2, emb)

    times_f = jnp.concatenate([time_1, time_2], axis=0).astype(jnp.float32)
    zero_col = jnp.zeros((B, 1), time_gap_1.dtype)
    tg1 = jnp.concatenate([time_gap_1[:, 1:], zero_col], axis=1)
    tg2 = jnp.concatenate([time_gap_2[:, 1:], zero_col], axis=1)
    tgt = jnp.concatenate([jnp.concatenate([tg1, tg2], axis=0), times_f],
                          axis=1).astype(jnp.bfloat16)
    lens = jnp.concatenate([l1, l2]).reshape(B2, 1)
    seq, fsum = _lstm_loss(time_w, time_b, tgt, lens, time_table,
                           W_ih.T.astype(jnp.bfloat16),
                           W_hh.T.astype(jnp.bfloat16),
                           (b_ih + b_hh).reshape(1, 4 * H),
                           time_v.reshape(1, H))

    outs, tl = _final(hang, lie, seq, emb_u, fsum, lens, fc_W[0:S],
                      fc_W[S:2 * S], fc_W[2 * S:2 * S + H],
                      fc_W[2 * S + H:], fc_b.reshape(1, 2))
    return (outs, tl.reshape(()))


# speedup vs baseline: 11.8783x; 1.0487x over previous
import functools

import jax
import jax.numpy as jnp
from jax import lax
from jax.experimental import pallas as pl
from jax.experimental.pallas import tpu as pltpu
from jax.experimental.pallas import tpu_sc as plsc

B = 256
S = 200
ED = 64
H = 128
B2 = 2 * B
TIMEN = 168

NW = 32
HALF_PER_W = B * S // NW
CHUNK = 800
U_PER_W = B2 // NW


def _sc_gather_loc_user(loc_idx, u_idx, loc_table, gat_embedding):
    @functools.partial(
        pl.kernel,
        out_type=[
            jax.ShapeDtypeStruct((B * S, 2 * ED), jnp.float32),
            jax.ShapeDtypeStruct((B, 2 * ED), jnp.float32),
        ],
        mesh=plsc.VectorSubcoreMesh(core_axis_name="c", subcore_axis_name="s"),
        scratch_types=[
            [pltpu.VMEM((CHUNK,), jnp.int32)] * 2,
            [pltpu.VMEM((CHUNK, ED), jnp.float32)] * 2,
            pltpu.VMEM((U_PER_W,), jnp.int32),
            pltpu.VMEM((U_PER_W, ED), jnp.float32),
            [pltpu.SemaphoreType.DMA] * 2,
            [pltpu.SemaphoreType.DMA] * 2,
            pltpu.SemaphoreType.DMA,
        ],
        compiler_params=pltpu.CompilerParams(use_tc_tiling_on_sc=False),
    )
    def gather_kernel(loc_idx_h, u_idx_h, loc_t_h, gat_h, e_out,
                      u_out, idx_v, rows_v, uidx_v, urows_v, gsems, wsems,
                      usem):
        wid = lax.axis_index("s") * 2 + lax.axis_index("c")
        ubase = wid * U_PER_W
        pltpu.sync_copy(u_idx_h.at[pl.ds(ubase, U_PER_W)], uidx_v)
        uh = pltpu.async_copy(gat_h.at[uidx_v], urows_v, usem)

        nch = HALF_PER_W // CHUNK
        work = [(B * S * h, h * ED, j)
                for h in (0, 1)
                for j in range(nch)]
        gh = [None, None]
        wh = [None, None]
        pend = [None, None]
        for j in range(len(work) + 1):
            if j < len(work):
                bb = j % 2
                if wh[bb] is not None:
                    wh[bb].wait()
                off, col, cj = work[j]
                base = wid * HALF_PER_W + cj * CHUNK
                pltpu.sync_copy(loc_idx_h.at[pl.ds(off + base, CHUNK)],
                                idx_v[bb])
                gh[bb] = pltpu.async_copy(loc_t_h.at[idx_v[bb]], rows_v[bb],
                                          gsems[bb])
                pend[bb] = (col, base)
            if j >= 1:
                pb = (j - 1) % 2
                gh[pb].wait()
                col, base = pend[pb]
                wh[pb] = pltpu.async_copy(
                    rows_v[pb],
                    e_out.at[pl.ds(base, CHUNK), pl.ds(col, ED)],
                    wsems[pb])
        for h in wh:
            if h is not None:
                h.wait()
        uh.wait()
        is_u2 = (wid >= NW // 2).astype(jnp.int32)
        urow = ubase - B * is_u2
        ucol = ED * is_u2
        pltpu.sync_copy(urows_v,
                        u_out.at[pl.ds(urow, U_PER_W), pl.ds(ucol, ED)])

    return gather_kernel(loc_idx, u_idx, loc_table, gat_embedding)


COS_BATCH = 8


def _cos_topmax(length_1, length_2, emb):
    def body(l1_ref, l2_ref, e_ref, hang_ref, lie_ref):
        b0 = pl.program_id(0) * COS_BATCH
        neg = jnp.float32(-jnp.inf)
        x = e_ref[...]
        e1a = x[:, 0:ED]
        e2a = x[:, ED:2 * ED]
        r1a = lax.rsqrt(jnp.sum(e1a * e1a, axis=1, keepdims=True))
        r2a = lax.rsqrt(jnp.sum(e2a * e2a, axis=1, keepdims=True))
        n1a = (e1a * r1a).astype(jnp.bfloat16)
        n2a = (e2a * r2a).astype(jnp.bfloat16)
        hang_cols = []
        lie_rows = []
        for i in range(COS_BATCH):
            cos = lax.dot_general(n1a[i * S:(i + 1) * S, :],
                                  n2a[i * S:(i + 1) * S, :],
                                  (((1,), (1,)), ((), ())),
                                  preferred_element_type=jnp.float32)
            l1 = l1_ref[b0 + i]
            l2 = l2_ref[b0 + i]
            row_id = lax.broadcasted_iota(jnp.int32, (S, S), 0)
            col_id = lax.broadcasted_iota(jnp.int32, (S, S), 1)
            hang_full = jnp.max(jnp.where(col_id < l2, cos, neg), axis=1,
                                keepdims=True)
            rmask = lax.broadcasted_iota(jnp.int32, (S, 1), 0) < l1
            hang_cols.append(jnp.where(rmask, hang_full, 0.0))
            lie_full = jnp.max(jnp.where(row_id < l1, cos, neg), axis=0,
                               keepdims=True)
            cmask = lax.broadcasted_iota(jnp.int32, (1, S), 1) < l2
            lie_rows.append(jnp.where(cmask, lie_full, 0.0))
        hang_ref[...] = jnp.concatenate(hang_cols, axis=1).T
        lie_ref[...] = jnp.concatenate(lie_rows, axis=0)

    return pl.pallas_call(
        body,
        grid=(B // COS_BATCH,),
        in_specs=[
            pl.BlockSpec(memory_space=pltpu.SMEM),
            pl.BlockSpec(memory_space=pltpu.SMEM),
            pl.BlockSpec((COS_BATCH * S, 2 * ED), lambda b: (b, 0)),
        ],
        out_specs=[
            pl.BlockSpec((COS_BATCH, S), lambda b: (b, 0)),
            pl.BlockSpec((COS_BATCH, S), lambda b: (b, 0)),
        ],
        out_shape=[
            jax.ShapeDtypeStruct((B, S), jnp.float32),
            jax.ShapeDtypeStruct((B, S), jnp.float32),
        ],
    )(length_1, length_2, emb)


LSTM_UNROLL = 10


def _lstm_loss(w_s, b_s, tgt, lens, time_table, W_ihT, W_hhT, b_tot, tv_row):
    def body(w_ref, b0_ref, tgt_ref, len_ref, tt_ref, wih_ref, whh_ref,
             bt_ref, tv_ref, seq_out, fs_out, h_s, c_s, seq_s, fs_s, proj_s):
        pid = pl.program_id(0)

        @pl.when(pid == 0)
        def _():
            h_s[...] = jnp.zeros_like(h_s)
            c_s[...] = jnp.zeros_like(c_s)
            seq_s[...] = jnp.zeros_like(seq_s)
            fs_s[...] = jnp.zeros_like(fs_s)
            proj_s[...] = lax.dot_general(
                tt_ref[...].astype(jnp.bfloat16), wih_ref[...],
                (((1,), (0,)), ((), ())),
                preferred_element_type=jnp.float32).astype(jnp.bfloat16)

        w = w_ref[0, 0]
        b0 = b0_ref[0, 0]
        lm1 = len_ref[...] - 1
        hb = h_s[...]
        c = c_s[...]
        seq = seq_s[...]
        fs = fs_s[...]
        for u in range(LSTM_UNROLL):
            step = pid * LSTM_UNROLL + u
            ri = lax.broadcasted_iota(jnp.int32, (2 * S, 2), 0)
            ci = lax.broadcasted_iota(jnp.int32, (2 * S, 2), 1)
            sel = (ri == step + S * ci).astype(jnp.bfloat16)
            tc2 = lax.dot_general(tgt_ref[...], sel, (((1,), (0,)), ((), ())),
                                  preferred_element_type=jnp.float32)
            tgv = tc2[:, 0:1]
            onehot = (tc2[:, 1:2].astype(jnp.int32)
                      == lax.broadcasted_iota(jnp.int32, (B2, TIMEN), 1)
                      ).astype(jnp.bfloat16)
            gates = (lax.dot_general(onehot, proj_s[...],
                                     (((1,), (0,)), ((), ())),
                                     preferred_element_type=jnp.float32)
                     + lax.dot_general(hb, whh_ref[...],
                                       (((1,), (0,)), ((), ())),
                                       preferred_element_type=jnp.float32)
                     + bt_ref[...])
            gi = jax.nn.sigmoid(gates[:, 0:H])
            gf = jax.nn.sigmoid(gates[:, H:2 * H])
            gg = jnp.tanh(gates[:, 2 * H:3 * H])
            go = jax.nn.sigmoid(gates[:, 3 * H:4 * H])
            c = gf * c + gi * gg
            h = go * jnp.tanh(c)
            hb = h.astype(jnp.bfloat16)

            p1 = jnp.sum(h * tv_ref[...], axis=1, keepdims=True)
            p2 = w * tgv
            f1 = (p1 + p2 + b0
                  + (jnp.exp(p1 + b0) - jnp.exp(p1 + p2 + b0)) / w)
            fs = fs + jnp.where(step < lm1, f1, 0.0)
            seq = jnp.where(lm1 == step, h, seq)
        h_s[...] = hb
        c_s[...] = c
        seq_s[...] = seq
        fs_s[...] = fs

        @pl.when(pid == S // LSTM_UNROLL - 1)
        def _():
            seq_out[...] = seq
            fs_out[...] = fs

    return pl.pallas_call(
        body,
        grid=(S // LSTM_UNROLL,),
        in_specs=[
            pl.BlockSpec(memory_space=pltpu.SMEM),
            pl.BlockSpec(memory_space=pltpu.SMEM),
            pl.BlockSpec((B2, 2 * S), lambda s: (0, 0)),
            pl.BlockSpec((B2, 1), lambda s: (0, 0)),
            pl.BlockSpec((TIMEN, ED), lambda s: (0, 0)),
            pl.BlockSpec((ED, 4 * H), lambda s: (0, 0)),
            pl.BlockSpec((H, 4 * H), lambda s: (0, 0)),
            pl.BlockSpec((1, 4 * H), lambda s: (0, 0)),
            pl.BlockSpec((1, H), lambda s: (0, 0)),
        ],
        out_specs=[
            pl.BlockSpec((B2, H), lambda s: (0, 0)),
            pl.BlockSpec((B2, 1), lambda s: (0, 0)),
        ],
        out_shape=[
            jax.ShapeDtypeStruct((B2, H), jnp.float32),
            jax.ShapeDtypeStruct((B2, 1), jnp.float32),
        ],
        scratch_shapes=[
            pltpu.VMEM((B2, H), jnp.bfloat16),
            pltpu.VMEM((B2, H), jnp.float32),
            pltpu.VMEM((B2, H), jnp.float32),
            pltpu.VMEM((B2, 1), jnp.float32),
            pltpu.VMEM((TIMEN, 4 * H), jnp.bfloat16),
        ],
    )(w_s, b_s, tgt, lens, time_table, W_ihT, W_hhT, b_tot, tv_row)


def _final(hang, lie, seq, u_emb, fsum, lens, w_hang, w_lie, w_time, w_u,
           fcb):
    def body(hang_ref, lie_ref, seq_ref, u_ref, fs_ref, len_ref, wh_ref,
             wl_ref, wt_ref, wu_ref, fb_ref, out_ref, tl_ref):
        th = jnp.tanh(seq_ref[0:B, :] * seq_ref[B:B2, :])
        tu = jnp.tanh(u_ref[:, 0:ED] * u_ref[:, ED:2 * ED])

        def dn(a, w):
            return lax.dot_general(a, w, (((1,), (0,)), ((), ())),
                                   preferred_element_type=jnp.float32)

        out_ref[...] = (dn(hang_ref[...], wh_ref[...])
                        + dn(lie_ref[...], wl_ref[...])
                        + dn(th, wt_ref[...])
                        + dn(tu, wu_ref[...])
                        + fb_ref[...])
        r = fs_ref[...] / (len_ref[...] - 1).astype(jnp.float32)
        tl_ref[...] = (-jnp.sum(r) / B).reshape(1, 1)

    return pl.pallas_call(
        body,
        out_shape=[
            jax.ShapeDtypeStruct((B, 2), jnp.float32),
            jax.ShapeDtypeStruct((1, 1), jnp.float32),
        ],
    )(hang, lie, seq, u_emb, fsum, lens, w_hang, w_lie, w_time, w_u, fcb)


def kernel(u1, u2, length_1, length_2, loc_1, loc_2, time_1, time_2,
           time_gap_1, time_gap_2, loc_table, time_table, W_ih, W_hh, b_ih,
           b_hh, time_v, time_w, time_b, gat_embedding, fc_W, fc_b):
    loc_idx = jnp.concatenate([loc_1.reshape(-1),
                               loc_2.reshape(-1)]).astype(jnp.int32)
    u_idx = jnp.concatenate([u1, u2]).astype(jnp.int32)
    emb, emb_u = _sc_gather_loc_user(loc_idx, u_idx, loc_table,
                                     gat_embedding)

    l1 = length_1.astype(jnp.int32)
    l2 = length_2.astype(jnp.int32)
    hang, lie = _cos_topmax(l1, l2, emb)

    times_f = jnp.concatenate([time_1, time_2], axis=0).astype(jnp.float32)
    zero_col = jnp.zeros((B, 1), time_gap_1.dtype)
    tg1 = jnp.concatenate([time_gap_1[:, 1:], zero_col], axis=1)
    tg2 = jnp.concatenate([time_gap_2[:, 1:], zero_col], axis=1)
    tgt = jnp.concatenate([jnp.concatenate([tg1, tg2], axis=0), times_f],
                          axis=1).astype(jnp.bfloat16)
    lens = jnp.concatenate([l1, l2]).reshape(B2, 1)
    seq, fsum = _lstm_loss(time_w, time_b, tgt, lens, time_table,
                           W_ih.T.astype(jnp.bfloat16),
                           W_hh.T.astype(jnp.bfloat16),
                           (b_ih + b_hh).reshape(1, 4 * H),
                           time_v.reshape(1, H))

    outs, tl = _final(hang, lie, seq, emb_u, fsum, lens, fc_W[0:S],
                      fc_W[S:2 * S], fc_W[2 * S:2 * S + H],
                      fc_W[2 * S + H:], fc_b.reshape(1, 2))
    return (outs, tl.reshape(()))
